# baseline (speedup 1.0000x reference)
import math

import jax
import jax.numpy as jnp
from jax.experimental import pallas as pl

N = 10000
DEPTH = 3
POOL_RATIO = 0.5


def _add_bias_pallas(x, b):
    def body(x_ref, b_ref, o_ref):
        o_ref[...] = x_ref[...] + b_ref[...]

    return pl.pallas_call(
        body,
        out_shape=jax.ShapeDtypeStruct(x.shape, x.dtype),
    )(x, jnp.broadcast_to(b, x.shape))


def _gcn(x, row, col, ew, W, b, n):
    loop = jnp.arange(n, dtype=row.dtype)
    r = jnp.concatenate([row, loop])
    c = jnp.concatenate([col, loop])
    w = jnp.concatenate([ew, jnp.ones((n,), x.dtype)])
    deg = jnp.zeros((n,), x.dtype).at[c].add(w)
    dinv = jnp.where(deg > 0, jax.lax.rsqrt(deg), 0.0)
    norm = dinv[r] * w * dinv[c]
    xw = x @ W
    out = jnp.zeros((n, W.shape[1]), x.dtype).at[c].add(xw[r] * norm[:, None])
    return out + b


def _pool(x, row, col, ew, p, k):
    n = x.shape[0]
    score = (x @ p) / jnp.linalg.norm(p)
    vals, perm = jax.lax.top_k(score, k)
    xn = x[perm] * jnp.tanh(vals)[:, None]
    kept = jnp.zeros((n,), dtype=bool).at[perm].set(True)
    remap = jnp.zeros((n,), dtype=row.dtype).at[perm].set(jnp.arange(k, dtype=row.dtype))
    m = kept[row] & kept[col]
    nr = jnp.where(m, remap[row], 0)
    nc = jnp.where(m, remap[col], 0)
    nw = jnp.where(m, ew, 0.0)
    return xn, nr, nc, nw, perm


def kernel(x, edge_index, down_w0, down_b0, down_w1, down_b1, down_w2, down_b2,
           down_w3, down_b3, pool_p0, pool_p1, pool_p2,
           up_w0, up_b0, up_w1, up_b1, up_w2, up_b2):
    params = {
        "down_w0": down_w0, "down_b0": down_b0,
        "down_w1": down_w1, "down_b1": down_b1,
        "down_w2": down_w2, "down_b2": down_b2,
        "down_w3": down_w3, "down_b3": down_b3,
        "pool_p0": pool_p0, "pool_p1": pool_p1, "pool_p2": pool_p2,
        "up_w0": up_w0, "up_b0": up_b0,
        "up_w1": up_w1, "up_b1": up_b1,
        "up_w2": up_w2, "up_b2": up_b2,
    }
    row, col = edge_index[0], edge_index[1]
    ew = jnp.ones((row.shape[0],), x.dtype)
    n = x.shape[0]
    x = jax.nn.relu(_gcn(x, row, col, ew, params["down_w0"], params["down_b0"], n))
    xs = [x]
    rows = [row]
    cols = [col]
    ews = [ew]
    perms = []
    for i in range(1, DEPTH + 1):
        k = int(math.ceil(POOL_RATIO * n))
        x, row, col, ew, perm = _pool(x, row, col, ew, params[f"pool_p{i-1}"], k)
        n = k
        x = jax.nn.relu(_gcn(x, row, col, ew, params[f"down_w{i}"], params[f"down_b{i}"], n))
        if i < DEPTH:
            xs.append(x)
            rows.append(row)
            cols.append(col)
            ews.append(ew)
        perms.append(perm)
    for i in range(DEPTH):
        j = DEPTH - 1 - i
        res = xs[j]
        up = jnp.zeros_like(res).at[perms[j]].set(x)
        x = res + up
        bias = params[f"up_b{i}"]
        x = _gcn(x, rows[j], cols[j], ews[j], params[f"up_w{i}"], jnp.zeros_like(bias), res.shape[0])
        x = _add_bias_pallas(x, bias)
        if i < DEPTH - 1:
            x = jax.nn.relu(x)
    return x


# speedup vs baseline: 32.5755x; 32.5755x over previous
import functools

import jax
import jax.numpy as jnp
from jax import lax
from jax.experimental import pallas as pl
from jax.experimental.pallas import tpu as pltpu
from jax.experimental.pallas import tpu_sc as plsc

N = 10000
NP = 10240
E = 320000
H = 128
DEPTH = 3
NW = 32
EPW = E // NW
CH = 100
NCH = EPW // CH
HH = H // 2
EPT = E // 16
NCT = EPT // CH
RPT = NP // 16

_mesh = plsc.VectorSubcoreMesh(core_axis_name="c", subcore_axis_name="s")


@functools.partial(
    pl.kernel,
    mesh=_mesh,
    compiler_params=pltpu.CompilerParams(use_tc_tiling_on_sc=False,
                                         needs_layout_passes=False),
    out_type=jax.ShapeDtypeStruct((NW, NP), jnp.float32),
    scratch_types=[
        pltpu.VMEM((NP,), jnp.float32),
        pltpu.VMEM((EPW,), jnp.int32),
        pltpu.VMEM((EPW,), jnp.int32),
        pltpu.VMEM((NP,), jnp.float32),
    ],
)
def _deg_sc(rows_hbm, cols_hbm, m_hbm, out_hbm, m_v, r_v, c_v, s_v):
    cid = lax.axis_index("c")
    sid = lax.axis_index("s")
    wid = cid * 16 + sid
    pltpu.sync_copy(m_hbm, m_v)
    pltpu.sync_copy(rows_hbm.at[wid], r_v)
    pltpu.sync_copy(cols_hbm.at[wid], c_v)
    z16 = jnp.zeros((16,), jnp.float32)

    def zbody(i, _):
        s_v[pl.ds(i * 16, 16)] = z16
        return 0

    lax.fori_loop(0, NP // 16, zbody, 0)

    def ebody(i, _):
        ridx = r_v[pl.ds(i * 16, 16)]
        cidx = c_v[pl.ds(i * 16, 16)]
        val = plsc.load_gather(m_v, [ridx])
        plsc.addupdate_scatter(s_v, [cidx], val)
        return 0

    lax.fori_loop(0, EPW // 16, ebody, 0)
    pltpu.sync_copy(s_v, out_hbm.at[wid])


@functools.partial(
    pl.kernel,
    mesh=_mesh,
    compiler_params=pltpu.CompilerParams(use_tc_tiling_on_sc=False,
                                         needs_layout_passes=False),
    out_type=jax.ShapeDtypeStruct((2 * NP, HH), jnp.float32),
    scratch_types=[
        pltpu.VMEM((NCT, CH), jnp.int32),
        pltpu.VMEM((NCT, CH), jnp.int32),
        pltpu.VMEM((CH, HH), jnp.float32),
        pltpu.VMEM((128, HH), jnp.float32),
        pltpu.VMEM_SHARED((NP, HH), jnp.float32),
        pltpu.SemaphoreType.DMA,
    ],
)
def _row_sc(y0_hbm, y1_hbm, rows_hbm, cols_hbm, z_hbm, out_hbm,
            r_v, c_v, rowbuf, zbuf, agg_sp, sem):
    cid = lax.axis_index("c")
    sid = lax.axis_index("s")
    pltpu.sync_copy(z_hbm, zbuf)
    base = sid * RPT
    for t in range(RPT // 128):
        pltpu.sync_copy(zbuf, agg_sp.at[pl.ds(base + t * 128, 128)])
    plsc.subcore_barrier()
    pltpu.sync_copy(rows_hbm.at[sid], r_v)
    pltpu.sync_copy(cols_hbm.at[sid], c_v)

    for c_static, y_hbm in ((0, y0_hbm), (1, y1_hbm)):
        @pl.when(cid == c_static)
        def _():
            def ebody(g, _):
                pltpu.async_copy(y_hbm.at[r_v.at[g]], rowbuf, sem).wait()
                pltpu.sync_copy(rowbuf, agg_sp.at[c_v.at[g]], add=True)
                return 0

            lax.fori_loop(0, NCT, ebody, 0)

    plsc.subcore_barrier()
    for t in range(RPT // 128):
        pltpu.sync_copy(agg_sp.at[pl.ds(base + t * 128, 128)], zbuf)
        pltpu.sync_copy(zbuf, out_hbm.at[pl.ds(cid * NP + base + t * 128, 128)])


def _dot(a, b):
    return jnp.dot(a, b, preferred_element_type=jnp.float32,
                   precision=lax.Precision.HIGHEST)


def _prep_first_body(x_ref, w_ref, sp_ref, m_ref, y_ref, dinv_ref):
    s = jnp.sum(sp_ref[...], axis=1, keepdims=True)
    dinv = m_ref[...] * lax.rsqrt(s + 1.0)
    y_ref[...] = dinv * _dot(x_ref[...], w_ref[...])
    dinv_ref[...] = dinv


def _prep_gated_body(x_ref, g_ref, w_ref, sp_ref, m_ref, y_ref, dinv_ref):
    s = jnp.sum(sp_ref[...], axis=1, keepdims=True)
    dinv = m_ref[...] * lax.rsqrt(s + 1.0)
    y_ref[...] = dinv * _dot(x_ref[...] * g_ref[...], w_ref[...])
    dinv_ref[...] = dinv


def _prep_up_body(res_ref, xd_ref, md_ref, w_ref, dinv_ref, y_ref):
    xin = res_ref[...] + xd_ref[...] * md_ref[...]
    y_ref[...] = dinv_ref[...] * _dot(xin, w_ref[...])


def _combine_body(relu, agg_ref, y_ref, dinv_ref, b_ref, o_ref):
    agg = jnp.concatenate([agg_ref[0:NP, :], agg_ref[NP:2 * NP, :]], axis=1)
    out = dinv_ref[...] * (agg + y_ref[...]) + b_ref[...]
    o_ref[...] = jnp.maximum(out, 0.0) if relu else out


def _score_body(h_ref, p_ref, o_ref):
    nrm = jnp.sqrt(jnp.sum(p_ref[...] * p_ref[...]))
    o_ref[...] = _dot(h_ref[...], p_ref[...]) / nrm


def _select_body(k, scr_ref, m_ref, sel_ref, gate_ref):
    scr = scr_ref[...]
    bits = lax.bitcast_convert_type(scr, jnp.uint32)
    okey = jnp.where(bits >= jnp.uint32(0x80000000), ~bits,
                     bits | jnp.uint32(0x80000000))
    skey = jnp.where(m_ref[...] > 0, okey, jnp.uint32(0))
    r_i = lax.broadcasted_iota(jnp.uint32, skey.shape, 0)
    c_i = lax.broadcasted_iota(jnp.uint32, skey.shape, 1)
    lo = jnp.uint32(16383) - (r_i * jnp.uint32(128) + c_i)

    def hib(t, carry):
        pref, rem = carry
        sh = 31 - t
        cand = pref | (jnp.uint32(1) << sh)
        c = jnp.sum(((skey >> sh) == (cand >> sh)).astype(jnp.int32))
        take = c >= rem
        return (jnp.where(take, cand, pref), jnp.where(take, rem, rem - c))

    th, rem = lax.fori_loop(0, 32, hib, (jnp.uint32(0), jnp.int32(k)))

    def lob(t, carry):
        plo, rem = carry
        sh = 13 - t
        cand = plo | (jnp.uint32(1) << sh)
        c = jnp.sum(((skey == th) & ((lo >> sh) == (cand >> sh)))
                    .astype(jnp.int32))
        take = c >= rem
        return (jnp.where(take, cand, plo), jnp.where(take, rem, rem - c))

    tl, _ = lax.fori_loop(0, 14, lob, (jnp.uint32(0), rem))
    sel = (skey > th) | ((skey == th) & (lo >= tl))
    self_f = sel.astype(jnp.float32)
    sel_ref[...] = self_f
    gate_ref[...] = jnp.tanh(scr) * self_f


def _tc_call(body, out_shapes, *args):
    return pl.pallas_call(
        body,
        out_shape=out_shapes,
    )(*args)


def _gcn_layer(x, gate, up_pair, w, b, dinv, sparts, m, rows_ch, cols_ch,
               zrow, relu):
    if up_pair is not None:
        xd, md = up_pair
        y = _tc_call(_prep_up_body,
                     jax.ShapeDtypeStruct((NP, H), jnp.float32),
                     x, xd, md, w, dinv)
    elif gate is None:
        y, dinv = _tc_call(_prep_first_body,
                           (jax.ShapeDtypeStruct((NP, H), jnp.float32),
                            jax.ShapeDtypeStruct((NP, 1), jnp.float32)),
                           x, w, sparts, m)
    else:
        y, dinv = _tc_call(_prep_gated_body,
                           (jax.ShapeDtypeStruct((NP, H), jnp.float32),
                            jax.ShapeDtypeStruct((NP, 1), jnp.float32)),
                           x, gate, w, sparts, m)
    y0 = lax.slice(y, (0, 0), (NP, HH))
    y1 = lax.slice(y, (0, HH), (NP, H))
    agg = _row_sc(y0, y1, rows_ch, cols_ch, zrow)
    out = _tc_call(functools.partial(_combine_body, relu),
                   jax.ShapeDtypeStruct((NP, H), jnp.float32),
                   agg, y, dinv, b)
    return out, dinv


def kernel(x, edge_index, down_w0, down_b0, down_w1, down_b1, down_w2,
           down_b2, down_w3, down_b3, pool_p0, pool_p1, pool_p2,
           up_w0, up_b0, up_w1, up_b1, up_w2, up_b2):
    f32 = jnp.float32
    rows = edge_index[0]
    cols = edge_index[1]
    rows_fl = rows.reshape(NW, EPW)
    cols_fl = cols.reshape(NW, EPW)
    rows_ch = rows.reshape(16, NCT, CH)
    cols_ch = cols.reshape(16, NCT, CH)
    zrow = jnp.zeros((128, HH), f32)

    xp = jnp.pad(x, ((0, NP - N), (0, 0)))
    m0f = jnp.zeros((NP,), f32).at[:N].set(1.0)
    down_w = [down_w0, down_w1, down_w2, down_w3]
    down_b = [down_b0.reshape(1, H), down_b1.reshape(1, H),
              down_b2.reshape(1, H), down_b3.reshape(1, H)]
    up_w = [up_w0, up_w1, up_w2]
    up_b = [up_b0.reshape(1, H), up_b1.reshape(1, H), up_b2.reshape(1, H)]
    pool_p = [pool_p0.reshape(H, 1), pool_p1.reshape(H, 1),
              pool_p2.reshape(H, 1)]

    def deg(mf):
        sparts = _deg_sc(rows_fl, cols_fl, mf)
        return sparts.T

    masks_f = [m0f]
    masks_c = [m0f.reshape(NP, 1)]
    sparts = deg(m0f)
    h, dinv0 = _gcn_layer(xp, None, None, down_w[0], down_b[0], None, sparts,
                          masks_c[0], rows_ch, cols_ch, zrow, relu=True)
    hs = [h]
    dinvs = [dinv0]
    n_act = N
    for i in range(1, DEPTH + 1):
        k = (n_act + 1) // 2
        scr = _tc_call(_score_body, jax.ShapeDtypeStruct((NP, 1), f32),
                       h, pool_p[i - 1])
        sel2, gate2 = _tc_call(
            functools.partial(_select_body, k),
            (jax.ShapeDtypeStruct((NP // 128, 128), f32),
             jax.ShapeDtypeStruct((NP // 128, 128), f32)),
            scr.reshape(NP // 128, 128), masks_f[-1].reshape(NP // 128, 128))
        mf = sel2.reshape(NP)
        gate = gate2.reshape(NP, 1)
        masks_f.append(mf)
        masks_c.append(mf.reshape(NP, 1))
        sparts = deg(mf)
        h, dinv = _gcn_layer(h, gate, None, down_w[i], down_b[i], None,
                             sparts, masks_c[i], rows_ch, cols_ch, zrow,
                             relu=True)
        if i < DEPTH:
            hs.append(h)
        dinvs.append(dinv)
        n_act = k

    for i in range(DEPTH):
        j = DEPTH - 1 - i
        h, _ = _gcn_layer(hs[j], None, (h, masks_c[j + 1]), up_w[i], up_b[i],
                          dinvs[j], None, None, rows_ch, cols_ch, zrow,
                          relu=(i < DEPTH - 1))
    return h[:N]


# speedup vs baseline: 45.0524x; 1.3830x over previous
import functools

import jax
import jax.numpy as jnp
from jax import lax
from jax.experimental import pallas as pl
from jax.experimental.pallas import tpu as pltpu
from jax.experimental.pallas import tpu_sc as plsc

N = 10000
NP = 10240
E = 320000
H = 128
DEPTH = 3
NW = 32
EPW = E // NW
CH = 125
NCH = EPW // CH
HH = H // 2
EPT = E // 16
NCT = EPT // CH
RPT = NP // 16

_mesh = plsc.VectorSubcoreMesh(core_axis_name="c", subcore_axis_name="s")


@functools.partial(
    pl.kernel,
    mesh=_mesh,
    compiler_params=pltpu.CompilerParams(use_tc_tiling_on_sc=False,
                                         needs_layout_passes=False),
    out_type=jax.ShapeDtypeStruct((NW, NP), jnp.float32),
    scratch_types=[
        pltpu.VMEM((NP,), jnp.float32),
        pltpu.VMEM((EPW,), jnp.int32),
        pltpu.VMEM((EPW,), jnp.int32),
        pltpu.VMEM((NP,), jnp.float32),
    ],
)
def _deg_sc(rows_hbm, cols_hbm, m_hbm, out_hbm, m_v, r_v, c_v, s_v):
    cid = lax.axis_index("c")
    sid = lax.axis_index("s")
    wid = cid * 16 + sid
    pltpu.sync_copy(m_hbm, m_v)
    pltpu.sync_copy(rows_hbm.at[wid], r_v)
    pltpu.sync_copy(cols_hbm.at[wid], c_v)
    z16 = jnp.zeros((16,), jnp.float32)

    def zbody(i, _):
        s_v[pl.ds(i * 16, 16)] = z16
        return 0

    lax.fori_loop(0, NP // 16, zbody, 0)

    def ebody(i, _):
        ridx = r_v[pl.ds(i * 16, 16)]
        cidx = c_v[pl.ds(i * 16, 16)]
        val = plsc.load_gather(m_v, [ridx])
        plsc.addupdate_scatter(s_v, [cidx], val)
        return 0

    lax.fori_loop(0, EPW // 16, ebody, 0)
    pltpu.sync_copy(s_v, out_hbm.at[wid])


@functools.partial(
    pl.kernel,
    mesh=_mesh,
    compiler_params=pltpu.CompilerParams(use_tc_tiling_on_sc=False,
                                         needs_layout_passes=False),
    out_type=jax.ShapeDtypeStruct((2 * NP, HH), jnp.float32),
    scratch_types=[
        pltpu.VMEM((NCT, CH), jnp.int32),
        pltpu.VMEM((NCT, CH), jnp.int32),
        pltpu.VMEM((CH, HH), jnp.float32),
        pltpu.VMEM((CH, HH), jnp.float32),
        pltpu.VMEM((128, HH), jnp.float32),
        pltpu.VMEM_SHARED((NP, HH), jnp.float32),
        pltpu.SemaphoreType.DMA,
        pltpu.SemaphoreType.DMA,
        pltpu.SemaphoreType.DMA,
        pltpu.SemaphoreType.DMA,
    ],
)
def _row_sc(y0_hbm, y1_hbm, rows_hbm, cols_hbm, z_hbm, out_hbm,
            r_v, c_v, buf0, buf1, zbuf, agg_sp, semg0, semg1, sems0, sems1):
    cid = lax.axis_index("c")
    sid = lax.axis_index("s")
    pltpu.sync_copy(z_hbm, zbuf)
    base = sid * RPT
    for t in range(RPT // 128):
        pltpu.sync_copy(zbuf, agg_sp.at[pl.ds(base + t * 128, 128)])
    plsc.subcore_barrier()
    pltpu.sync_copy(rows_hbm.at[sid], r_v)
    pltpu.sync_copy(cols_hbm.at[sid], c_v)

    for c_static, y_hbm in ((0, y0_hbm), (1, y1_hbm)):
        @pl.when(cid == c_static)
        def _():
            pltpu.async_copy(y_hbm.at[r_v.at[0]], buf0, semg0)

            def pair(gp, _):
                g0 = gp * 2
                pltpu.async_copy(y_hbm.at[r_v.at[g0 + 1]], buf1, semg1)
                pltpu.make_async_copy(y_hbm.at[r_v.at[g0]], buf0, semg0).wait()
                s0 = pltpu.async_copy(buf0, agg_sp.at[c_v.at[g0]], sems0,
                                      add=True)
                pltpu.make_async_copy(y_hbm.at[r_v.at[g0 + 1]], buf1,
                                      semg1).wait()
                s1 = pltpu.async_copy(buf1, agg_sp.at[c_v.at[g0 + 1]], sems1,
                                      add=True)
                s0.wait()

                @pl.when(g0 + 2 < NCT)
                def _():
                    pltpu.async_copy(y_hbm.at[r_v.at[g0 + 2]], buf0, semg0)

                s1.wait()
                return 0

            lax.fori_loop(0, NCT // 2, pair, 0)

    plsc.subcore_barrier()
    for t in range(RPT // 128):
        pltpu.sync_copy(agg_sp.at[pl.ds(base + t * 128, 128)], zbuf)
        pltpu.sync_copy(zbuf, out_hbm.at[pl.ds(cid * NP + base + t * 128, 128)])


def _dot(a, b):
    return jnp.dot(a, b, preferred_element_type=jnp.float32,
                   precision=lax.Precision.HIGHEST)


def _prep_first_body(x_ref, w_ref, sp_ref, m_ref, y_ref, dinv_ref):
    s = jnp.sum(sp_ref[...], axis=1, keepdims=True)
    dinv = m_ref[...] * lax.rsqrt(s + 1.0)
    y_ref[...] = dinv * _dot(x_ref[...], w_ref[...])
    dinv_ref[...] = dinv


def _prep_gated_body(x_ref, g_ref, w_ref, sp_ref, m_ref, y_ref, dinv_ref):
    s = jnp.sum(sp_ref[...], axis=1, keepdims=True)
    dinv = m_ref[...] * lax.rsqrt(s + 1.0)
    y_ref[...] = dinv * _dot(x_ref[...] * g_ref[...], w_ref[...])
    dinv_ref[...] = dinv


def _prep_up_body(res_ref, xd_ref, md_ref, w_ref, dinv_ref, y_ref):
    xin = res_ref[...] + xd_ref[...] * md_ref[...]
    y_ref[...] = dinv_ref[...] * _dot(xin, w_ref[...])


def _combine_body(relu, agg_ref, y_ref, dinv_ref, b_ref, o_ref):
    agg = jnp.concatenate([agg_ref[0:NP, :], agg_ref[NP:2 * NP, :]], axis=1)
    out = dinv_ref[...] * (agg + y_ref[...]) + b_ref[...]
    o_ref[...] = jnp.maximum(out, 0.0) if relu else out


def _score_body(h_ref, p_ref, o_ref):
    nrm = jnp.sqrt(jnp.sum(p_ref[...] * p_ref[...]))
    o_ref[...] = _dot(h_ref[...], p_ref[...]) / nrm


def _select_body(k, scr_ref, m_ref, sel_ref, gate_ref):
    scr = scr_ref[...]
    bits = lax.bitcast_convert_type(scr, jnp.uint32)
    okey = jnp.where(bits >= jnp.uint32(0x80000000), ~bits,
                     bits | jnp.uint32(0x80000000))
    skey = jnp.where(m_ref[...] > 0, okey, jnp.uint32(0))
    r_i = lax.broadcasted_iota(jnp.uint32, skey.shape, 0)
    c_i = lax.broadcasted_iota(jnp.uint32, skey.shape, 1)
    lo = jnp.uint32(16383) - (r_i * jnp.uint32(128) + c_i)

    def hib(t, carry):
        pref, rem = carry
        sh = 31 - t
        cand = pref | (jnp.uint32(1) << sh)
        c = jnp.sum(((skey >> sh) == (cand >> sh)).astype(jnp.int32))
        take = c >= rem
        return (jnp.where(take, cand, pref), jnp.where(take, rem, rem - c))

    th, rem = lax.fori_loop(0, 32, hib, (jnp.uint32(0), jnp.int32(k)))

    def lob(t, carry):
        plo, rem = carry
        sh = 13 - t
        cand = plo | (jnp.uint32(1) << sh)
        c = jnp.sum(((skey == th) & ((lo >> sh) == (cand >> sh)))
                    .astype(jnp.int32))
        take = c >= rem
        return (jnp.where(take, cand, plo), jnp.where(take, rem, rem - c))

    tl, _ = lax.fori_loop(0, 14, lob, (jnp.uint32(0), rem))
    sel = (skey > th) | ((skey == th) & (lo >= tl))
    self_f = sel.astype(jnp.float32)
    sel_ref[...] = self_f
    gate_ref[...] = jnp.tanh(scr) * self_f


def _tc_call(body, out_shapes, *args):
    return pl.pallas_call(
        body,
        out_shape=out_shapes,
    )(*args)


def _gcn_layer(x, gate, up_pair, w, b, dinv, sparts, m, rows_ch, cols_ch,
               zrow, relu):
    if up_pair is not None:
        xd, md = up_pair
        y = _tc_call(_prep_up_body,
                     jax.ShapeDtypeStruct((NP, H), jnp.float32),
                     x, xd, md, w, dinv)
    elif gate is None:
        y, dinv = _tc_call(_prep_first_body,
                           (jax.ShapeDtypeStruct((NP, H), jnp.float32),
                            jax.ShapeDtypeStruct((NP, 1), jnp.float32)),
                           x, w, sparts, m)
    else:
        y, dinv = _tc_call(_prep_gated_body,
                           (jax.ShapeDtypeStruct((NP, H), jnp.float32),
                            jax.ShapeDtypeStruct((NP, 1), jnp.float32)),
                           x, gate, w, sparts, m)
    y0 = lax.slice(y, (0, 0), (NP, HH))
    y1 = lax.slice(y, (0, HH), (NP, H))
    agg = _row_sc(y0, y1, rows_ch, cols_ch, zrow)
    out = _tc_call(functools.partial(_combine_body, relu),
                   jax.ShapeDtypeStruct((NP, H), jnp.float32),
                   agg, y, dinv, b)
    return out, dinv


def kernel(x, edge_index, down_w0, down_b0, down_w1, down_b1, down_w2,
           down_b2, down_w3, down_b3, pool_p0, pool_p1, pool_p2,
           up_w0, up_b0, up_w1, up_b1, up_w2, up_b2):
    f32 = jnp.float32
    rows = edge_index[0]
    cols = edge_index[1]
    rows_fl = rows.reshape(NW, EPW)
    cols_fl = cols.reshape(NW, EPW)
    rows_ch = rows.reshape(16, NCT, CH)
    cols_ch = cols.reshape(16, NCT, CH)
    zrow = jnp.zeros((128, HH), f32)

    xp = jnp.pad(x, ((0, NP - N), (0, 0)))
    m0f = jnp.zeros((NP,), f32).at[:N].set(1.0)
    down_w = [down_w0, down_w1, down_w2, down_w3]
    down_b = [down_b0.reshape(1, H), down_b1.reshape(1, H),
              down_b2.reshape(1, H), down_b3.reshape(1, H)]
    up_w = [up_w0, up_w1, up_w2]
    up_b = [up_b0.reshape(1, H), up_b1.reshape(1, H), up_b2.reshape(1, H)]
    pool_p = [pool_p0.reshape(H, 1), pool_p1.reshape(H, 1),
              pool_p2.reshape(H, 1)]

    def deg(mf):
        sparts = _deg_sc(rows_fl, cols_fl, mf)
        return sparts.T

    masks_f = [m0f]
    masks_c = [m0f.reshape(NP, 1)]
    sparts = deg(m0f)
    h, dinv0 = _gcn_layer(xp, None, None, down_w[0], down_b[0], None, sparts,
                          masks_c[0], rows_ch, cols_ch, zrow, relu=True)
    hs = [h]
    dinvs = [dinv0]
    n_act = N
    for i in range(1, DEPTH + 1):
        k = (n_act + 1) // 2
        scr = _tc_call(_score_body, jax.ShapeDtypeStruct((NP, 1), f32),
                       h, pool_p[i - 1])
        sel2, gate2 = _tc_call(
            functools.partial(_select_body, k),
            (jax.ShapeDtypeStruct((NP // 128, 128), f32),
             jax.ShapeDtypeStruct((NP // 128, 128), f32)),
            scr.reshape(NP // 128, 128), masks_f[-1].reshape(NP // 128, 128))
        mf = sel2.reshape(NP)
        gate = gate2.reshape(NP, 1)
        masks_f.append(mf)
        masks_c.append(mf.reshape(NP, 1))
        sparts = deg(mf)
        h, dinv = _gcn_layer(h, gate, None, down_w[i], down_b[i], None,
                             sparts, masks_c[i], rows_ch, cols_ch, zrow,
                             relu=True)
        if i < DEPTH:
            hs.append(h)
        dinvs.append(dinv)
        n_act = k

    for i in range(DEPTH):
        j = DEPTH - 1 - i
        h, _ = _gcn_layer(hs[j], None, (h, masks_c[j + 1]), up_w[i], up_b[i],
                          dinvs[j], None, None, rows_ch, cols_ch, zrow,
                          relu=(i < DEPTH - 1))
    return h[:N]


# speedup vs baseline: 52.2373x; 1.1595x over previous
import functools

import jax
import jax.numpy as jnp
from jax import lax
from jax.experimental import pallas as pl
from jax.experimental.pallas import tpu as pltpu
from jax.experimental.pallas import tpu_sc as plsc

N = 10000
NP = 10240
E = 320000
H = 128
DEPTH = 3
NW = 32
EPW = E // NW
CH = 125
NCH = EPW // CH
HH = H // 2
EPT = E // 16
NCT = EPT // CH
RPT = NP // 16
CAP = EPT + 256
CAPC = EPT // CH
PAIR = 2 * CH

_mesh = plsc.VectorSubcoreMesh(core_axis_name="c", subcore_axis_name="s")


@functools.partial(
    pl.kernel,
    mesh=_mesh,
    compiler_params=pltpu.CompilerParams(use_tc_tiling_on_sc=False,
                                         needs_layout_passes=False),
    out_type=jax.ShapeDtypeStruct((NW, NP), jnp.float32),
    scratch_types=[
        pltpu.VMEM((NP,), jnp.float32),
        pltpu.VMEM((EPW,), jnp.int32),
        pltpu.VMEM((EPW,), jnp.int32),
        pltpu.VMEM((NP,), jnp.float32),
    ],
)
def _deg_sc(rows_hbm, cols_hbm, m_hbm, out_hbm, m_v, r_v, c_v, s_v):
    cid = lax.axis_index("c")
    sid = lax.axis_index("s")
    wid = cid * 16 + sid
    pltpu.sync_copy(m_hbm, m_v)
    pltpu.sync_copy(rows_hbm.at[wid], r_v)
    pltpu.sync_copy(cols_hbm.at[wid], c_v)
    z16 = jnp.zeros((16,), jnp.float32)

    def zbody(i, _):
        s_v[pl.ds(i * 16, 16)] = z16
        return 0

    lax.fori_loop(0, NP // 16, zbody, 0)

    def ebody(i, _):
        ridx = r_v[pl.ds(i * 16, 16)]
        cidx = c_v[pl.ds(i * 16, 16)]
        val = plsc.load_gather(m_v, [ridx])
        plsc.addupdate_scatter(s_v, [cidx], val)
        return 0

    lax.fori_loop(0, EPW // 16, ebody, 0)
    pltpu.sync_copy(s_v, out_hbm.at[wid])


@functools.partial(
    pl.kernel,
    mesh=_mesh,
    compiler_params=pltpu.CompilerParams(use_tc_tiling_on_sc=False,
                                         needs_layout_passes=False),
    out_type=jax.ShapeDtypeStruct((2 * NP, HH), jnp.float32),
    scratch_types=[
        pltpu.VMEM((NCT, CH), jnp.int32),
        pltpu.VMEM((NCT, CH), jnp.int32),
        pltpu.VMEM((CH, HH), jnp.float32),
        pltpu.VMEM((CH, HH), jnp.float32),
        pltpu.VMEM((128, HH), jnp.float32),
        pltpu.VMEM_SHARED((NP, HH), jnp.float32),
        pltpu.SemaphoreType.DMA,
        pltpu.SemaphoreType.DMA,
        pltpu.SemaphoreType.DMA,
        pltpu.SemaphoreType.DMA,
    ],
)
def _row_sc(y0_hbm, y1_hbm, rows_hbm, cols_hbm, z_hbm, out_hbm,
            r_v, c_v, buf0, buf1, zbuf, agg_sp, semg0, semg1, sems0, sems1):
    cid = lax.axis_index("c")
    sid = lax.axis_index("s")
    pltpu.sync_copy(z_hbm, zbuf)
    base = sid * RPT
    for t in range(RPT // 128):
        pltpu.sync_copy(zbuf, agg_sp.at[pl.ds(base + t * 128, 128)])
    plsc.subcore_barrier()
    pltpu.sync_copy(rows_hbm.at[sid], r_v)
    pltpu.sync_copy(cols_hbm.at[sid], c_v)

    for c_static, y_hbm in ((0, y0_hbm), (1, y1_hbm)):
        @pl.when(cid == c_static)
        def _():
            pltpu.async_copy(y_hbm.at[r_v.at[0]], buf0, semg0)

            def pair(gp, _):
                g0 = gp * 2
                pltpu.async_copy(y_hbm.at[r_v.at[g0 + 1]], buf1, semg1)
                pltpu.make_async_copy(y_hbm.at[r_v.at[g0]], buf0, semg0).wait()
                s0 = pltpu.async_copy(buf0, agg_sp.at[c_v.at[g0]], sems0,
                                      add=True)
                pltpu.make_async_copy(y_hbm.at[r_v.at[g0 + 1]], buf1,
                                      semg1).wait()
                s1 = pltpu.async_copy(buf1, agg_sp.at[c_v.at[g0 + 1]], sems1,
                                      add=True)
                s0.wait()

                @pl.when(g0 + 2 < NCT)
                def _():
                    pltpu.async_copy(y_hbm.at[r_v.at[g0 + 2]], buf0, semg0)

                s1.wait()
                return 0

            lax.fori_loop(0, NCT // 2, pair, 0)

    plsc.subcore_barrier()
    for t in range(RPT // 128):
        pltpu.sync_copy(agg_sp.at[pl.ds(base + t * 128, 128)], zbuf)
        pltpu.sync_copy(zbuf, out_hbm.at[pl.ds(cid * NP + base + t * 128, 128)])


@functools.partial(
    pl.kernel,
    mesh=_mesh,
    compiler_params=pltpu.CompilerParams(use_tc_tiling_on_sc=False,
                                         needs_layout_passes=False),
    out_type=(jax.ShapeDtypeStruct((16, CAP), jnp.int32),
              jax.ShapeDtypeStruct((16, CAP), jnp.int32),
              jax.ShapeDtypeStruct((16, 16), jnp.int32)),
    scratch_types=[
        pltpu.VMEM((NP,), jnp.float32),
        pltpu.VMEM((EPT,), jnp.int32),
        pltpu.VMEM((EPT,), jnp.int32),
        pltpu.VMEM((CAP,), jnp.int32),
        pltpu.VMEM((CAP,), jnp.int32),
    ],
)
def _compact_sc(rows_hbm, cols_hbm, m_hbm, crows_hbm, ccols_hbm, counts_hbm,
                m_v, r_v, c_v, cr_v, cc_v):
    cid = lax.axis_index("c")
    sid = lax.axis_index("s")

    @pl.when(cid == 0)
    def _():
        pltpu.sync_copy(m_hbm, m_v)
        pltpu.sync_copy(rows_hbm.at[sid], r_v)
        pltpu.sync_copy(cols_hbm.at[sid], c_v)

        def ebody(i, off):
            rv = r_v[pl.ds(i * 16, 16)]
            cv = c_v[pl.ds(i * 16, 16)]
            mr = plsc.load_gather(m_v, [rv])
            mc = plsc.load_gather(m_v, [cv])
            keep = (mr > 0.0) & (mc > 0.0)
            plsc.store_compressed(cr_v.at[pl.ds(off, 16)], rv, mask=keep)
            plsc.store_compressed(cc_v.at[pl.ds(off, 16)], cv, mask=keep)
            nkeep = plsc.all_reduce_population_count(keep)
            return off + jnp.max(nkeep)

        cnt = lax.fori_loop(0, EPT // 16, ebody, jnp.int32(0))
        padv = jnp.full((16,), N, jnp.int32)
        for t in range(16):
            cr_v[pl.ds(cnt + t * 16, 16)] = padv
            cc_v[pl.ds(cnt + t * 16, 16)] = padv
        cntp = ((cnt + PAIR - 1) // PAIR) * PAIR
        pltpu.sync_copy(cr_v, crows_hbm.at[sid])
        pltpu.sync_copy(cc_v, ccols_hbm.at[sid])
        cr_v[pl.ds(CAP - 16, 16)] = jnp.full((16,), cntp, jnp.int32)
        pltpu.sync_copy(cr_v.at[pl.ds(CAP - 16, 16)], counts_hbm.at[sid])


@functools.partial(
    pl.kernel,
    mesh=_mesh,
    compiler_params=pltpu.CompilerParams(use_tc_tiling_on_sc=False,
                                         needs_layout_passes=False),
    out_type=jax.ShapeDtypeStruct((2 * NP, HH), jnp.float32),
    scratch_types=[
        pltpu.VMEM((CAPC, CH), jnp.int32),
        pltpu.VMEM((CAPC, CH), jnp.int32),
        pltpu.VMEM((16,), jnp.int32),
        pltpu.VMEM((CH, HH), jnp.float32),
        pltpu.VMEM((CH, HH), jnp.float32),
        pltpu.VMEM((128, HH), jnp.float32),
        pltpu.VMEM_SHARED((NP, HH), jnp.float32),
        pltpu.SemaphoreType.DMA,
        pltpu.SemaphoreType.DMA,
        pltpu.SemaphoreType.DMA,
        pltpu.SemaphoreType.DMA,
    ],
)
def _row_dyn_sc(y0_hbm, y1_hbm, crows_hbm, ccols_hbm, counts_hbm, z_hbm,
                out_hbm, r_v, c_v, cnt_vv, buf0, buf1, zbuf, agg_sp,
                semg0, semg1, sems0, sems1):
    cid = lax.axis_index("c")
    sid = lax.axis_index("s")
    pltpu.sync_copy(z_hbm, zbuf)
    base = sid * RPT
    for t in range(RPT // 128):
        pltpu.sync_copy(zbuf, agg_sp.at[pl.ds(base + t * 128, 128)])
    plsc.subcore_barrier()
    pltpu.sync_copy(crows_hbm.at[sid], r_v)
    pltpu.sync_copy(ccols_hbm.at[sid], c_v)
    pltpu.sync_copy(counts_hbm.at[sid], cnt_vv)
    nch = jnp.max(cnt_vv[...]) // CH

    for c_static, y_hbm in ((0, y0_hbm), (1, y1_hbm)):
        @pl.when((cid == c_static) & (nch > 0))
        def _():
            pltpu.async_copy(y_hbm.at[r_v.at[0]], buf0, semg0)

            def pair(gp, _):
                g0 = gp * 2
                pltpu.async_copy(y_hbm.at[r_v.at[g0 + 1]], buf1, semg1)
                pltpu.make_async_copy(y_hbm.at[r_v.at[g0]], buf0, semg0).wait()
                s0 = pltpu.async_copy(buf0, agg_sp.at[c_v.at[g0]], sems0,
                                      add=True)
                pltpu.make_async_copy(y_hbm.at[r_v.at[g0 + 1]], buf1,
                                      semg1).wait()
                s1 = pltpu.async_copy(buf1, agg_sp.at[c_v.at[g0 + 1]], sems1,
                                      add=True)
                s0.wait()

                @pl.when(g0 + 2 < nch)
                def _():
                    pltpu.async_copy(y_hbm.at[r_v.at[g0 + 2]], buf0, semg0)

                s1.wait()
                return 0

            lax.fori_loop(0, nch // 2, pair, 0)

    plsc.subcore_barrier()
    for t in range(RPT // 128):
        pltpu.sync_copy(agg_sp.at[pl.ds(base + t * 128, 128)], zbuf)
        pltpu.sync_copy(zbuf, out_hbm.at[pl.ds(cid * NP + base + t * 128, 128)])


def _dot(a, b):
    return jnp.dot(a, b, preferred_element_type=jnp.float32,
                   precision=lax.Precision.HIGHEST)


def _prep_first_body(x_ref, w_ref, sp_ref, m_ref, y_ref, dinv_ref):
    s = jnp.sum(sp_ref[...], axis=1, keepdims=True)
    dinv = m_ref[...] * lax.rsqrt(s + 1.0)
    y_ref[...] = dinv * _dot(x_ref[...], w_ref[...])
    dinv_ref[...] = dinv


def _prep_gated_body(x_ref, g_ref, w_ref, sp_ref, m_ref, y_ref, dinv_ref):
    s = jnp.sum(sp_ref[...], axis=1, keepdims=True)
    dinv = m_ref[...] * lax.rsqrt(s + 1.0)
    y_ref[...] = dinv * _dot(x_ref[...] * g_ref[...], w_ref[...])
    dinv_ref[...] = dinv


def _prep_up_body(res_ref, xd_ref, md_ref, w_ref, dinv_ref, y_ref):
    xin = res_ref[...] + xd_ref[...] * md_ref[...]
    y_ref[...] = dinv_ref[...] * _dot(xin, w_ref[...])


def _combine_body(relu, agg_ref, y_ref, dinv_ref, b_ref, o_ref):
    agg = jnp.concatenate([agg_ref[0:NP, :], agg_ref[NP:2 * NP, :]], axis=1)
    out = dinv_ref[...] * (agg + y_ref[...]) + b_ref[...]
    o_ref[...] = jnp.maximum(out, 0.0) if relu else out


def _score_body(h_ref, p_ref, o_ref):
    nrm = jnp.sqrt(jnp.sum(p_ref[...] * p_ref[...]))
    o_ref[...] = _dot(h_ref[...], p_ref[...]) / nrm


def _select_body(k, scr_ref, m_ref, sel_ref, gate_ref):
    scr = scr_ref[...]
    bits = lax.bitcast_convert_type(scr, jnp.uint32)
    okey = jnp.where(bits >= jnp.uint32(0x80000000), ~bits,
                     bits | jnp.uint32(0x80000000))
    skey = jnp.where(m_ref[...] > 0, okey, jnp.uint32(0))
    r_i = lax.broadcasted_iota(jnp.uint32, skey.shape, 0)
    c_i = lax.broadcasted_iota(jnp.uint32, skey.shape, 1)
    lo = jnp.uint32(16383) - (r_i * jnp.uint32(128) + c_i)

    def hib(t, carry):
        pref, rem = carry
        sh = 31 - t
        cand = pref | (jnp.uint32(1) << sh)
        c = jnp.sum(((skey >> sh) == (cand >> sh)).astype(jnp.int32))
        take = c >= rem
        return (jnp.where(take, cand, pref), jnp.where(take, rem, rem - c))

    th, rem = lax.fori_loop(0, 32, hib, (jnp.uint32(0), jnp.int32(k)))

    def lob(t, carry):
        plo, rem = carry
        sh = 13 - t
        cand = plo | (jnp.uint32(1) << sh)
        c = jnp.sum(((skey == th) & ((lo >> sh) == (cand >> sh)))
                    .astype(jnp.int32))
        take = c >= rem
        return (jnp.where(take, cand, plo), jnp.where(take, rem, rem - c))

    tl, _ = lax.fori_loop(0, 14, lob, (jnp.uint32(0), rem))
    sel = (skey > th) | ((skey == th) & (lo >= tl))
    self_f = sel.astype(jnp.float32)
    sel_ref[...] = self_f
    gate_ref[...] = jnp.tanh(scr) * self_f


def _tc_call(body, out_shapes, *args):
    return pl.pallas_call(
        body,
        out_shape=out_shapes,
    )(*args)


def _gcn_layer(x, gate, up_pair, w, b, dinv, sparts, m, edges, zrow, relu):
    if up_pair is not None:
        xd, md = up_pair
        y = _tc_call(_prep_up_body,
                     jax.ShapeDtypeStruct((NP, H), jnp.float32),
                     x, xd, md, w, dinv)
    elif gate is None:
        y, dinv = _tc_call(_prep_first_body,
                           (jax.ShapeDtypeStruct((NP, H), jnp.float32),
                            jax.ShapeDtypeStruct((NP, 1), jnp.float32)),
                           x, w, sparts, m)
    else:
        y, dinv = _tc_call(_prep_gated_body,
                           (jax.ShapeDtypeStruct((NP, H), jnp.float32),
                            jax.ShapeDtypeStruct((NP, 1), jnp.float32)),
                           x, gate, w, sparts, m)
    y0 = lax.slice(y, (0, 0), (NP, HH))
    y1 = lax.slice(y, (0, HH), (NP, H))
    if len(edges) == 2:
        agg = _row_sc(y0, y1, edges[0], edges[1], zrow)
    else:
        agg = _row_dyn_sc(y0, y1, edges[0], edges[1], edges[2], zrow)
    out = _tc_call(functools.partial(_combine_body, relu),
                   jax.ShapeDtypeStruct((NP, H), jnp.float32),
                   agg, y, dinv, b)
    return out, dinv


def kernel(x, edge_index, down_w0, down_b0, down_w1, down_b1, down_w2,
           down_b2, down_w3, down_b3, pool_p0, pool_p1, pool_p2,
           up_w0, up_b0, up_w1, up_b1, up_w2, up_b2):
    f32 = jnp.float32
    rows = edge_index[0]
    cols = edge_index[1]
    rows_fl = rows.reshape(NW, EPW)
    cols_fl = cols.reshape(NW, EPW)
    rows_ch = rows.reshape(16, NCT, CH)
    cols_ch = cols.reshape(16, NCT, CH)
    zrow = jnp.zeros((128, HH), f32)

    xp = jnp.pad(x, ((0, NP - N), (0, 0)))
    m0f = jnp.zeros((NP,), f32).at[:N].set(1.0)
    down_w = [down_w0, down_w1, down_w2, down_w3]
    down_b = [down_b0.reshape(1, H), down_b1.reshape(1, H),
              down_b2.reshape(1, H), down_b3.reshape(1, H)]
    up_w = [up_w0, up_w1, up_w2]
    up_b = [up_b0.reshape(1, H), up_b1.reshape(1, H), up_b2.reshape(1, H)]
    pool_p = [pool_p0.reshape(H, 1), pool_p1.reshape(H, 1),
              pool_p2.reshape(H, 1)]

    def deg(mf):
        sparts = _deg_sc(rows_fl, cols_fl, mf)
        return sparts.T

    full_edges = (rows_ch, cols_ch)
    masks_f = [m0f]
    masks_c = [m0f.reshape(NP, 1)]
    sparts = deg(m0f)
    h, dinv0 = _gcn_layer(xp, None, None, down_w[0], down_b[0], None, sparts,
                          masks_c[0], full_edges, zrow, relu=True)
    hs = [h]
    dinvs = [dinv0]
    n_act = N
    comp_edges = None
    for i in range(1, DEPTH + 1):
        k = (n_act + 1) // 2
        scr = _tc_call(_score_body, jax.ShapeDtypeStruct((NP, 1), f32),
                       h, pool_p[i - 1])
        sel2, gate2 = _tc_call(
            functools.partial(_select_body, k),
            (jax.ShapeDtypeStruct((NP // 128, 128), f32),
             jax.ShapeDtypeStruct((NP // 128, 128), f32)),
            scr.reshape(NP // 128, 128), masks_f[-1].reshape(NP // 128, 128))
        mf = sel2.reshape(NP)
        gate = gate2.reshape(NP, 1)
        masks_f.append(mf)
        masks_c.append(mf.reshape(NP, 1))
        if i == 1:
            rows_t = rows.reshape(16, EPT)
            cols_t = cols.reshape(16, EPT)
            cr, cc, cnts = _compact_sc(rows_t, cols_t, mf)
            cr3 = lax.slice(cr, (0, 0), (16, CAPC * CH)).reshape(16, CAPC, CH)
            cc3 = lax.slice(cc, (0, 0), (16, CAPC * CH)).reshape(16, CAPC, CH)
            comp_edges = (cr3, cc3, cnts)
        sparts = deg(mf)
        h, dinv = _gcn_layer(h, gate, None, down_w[i], down_b[i], None,
                             sparts, masks_c[i], comp_edges, zrow,
                             relu=True)
        if i < DEPTH:
            hs.append(h)
        dinvs.append(dinv)
        n_act = k

    for i in range(DEPTH):
        j = DEPTH - 1 - i
        edges = full_edges if j == 0 else comp_edges
        h, _ = _gcn_layer(hs[j], None, (h, masks_c[j + 1]), up_w[i], up_b[i],
                          dinvs[j], None, None, edges, zrow,
                          relu=(i < DEPTH - 1))
    return h[:N]


# speedup vs baseline: 53.9948x; 1.0336x over previous
import functools

import jax
import jax.numpy as jnp
from jax import lax
from jax.experimental import pallas as pl
from jax.experimental.pallas import tpu as pltpu
from jax.experimental.pallas import tpu_sc as plsc

N = 10000
NP = 10240
E = 320000
H = 128
DEPTH = 3
NW = 32
EPW = E // NW
CH = 125
NCH = EPW // CH
HH = H // 2
EPT = E // 16
NCT = EPT // CH
RPT = NP // 16
CAP = EPT + 256
CAPC = EPT // CH
PAIR = 2 * CH

_mesh = plsc.VectorSubcoreMesh(core_axis_name="c", subcore_axis_name="s")


@functools.partial(
    pl.kernel,
    mesh=_mesh,
    compiler_params=pltpu.CompilerParams(use_tc_tiling_on_sc=False,
                                         needs_layout_passes=False),
    out_type=jax.ShapeDtypeStruct((NW, NP), jnp.float32),
    scratch_types=[
        pltpu.VMEM((NP,), jnp.float32),
        pltpu.VMEM((EPW,), jnp.int32),
        pltpu.VMEM((EPW,), jnp.int32),
        pltpu.VMEM((NP,), jnp.float32),
    ],
)
def _deg_sc(rows_hbm, cols_hbm, m_hbm, out_hbm, m_v, r_v, c_v, s_v):
    cid = lax.axis_index("c")
    sid = lax.axis_index("s")
    wid = cid * 16 + sid
    pltpu.sync_copy(m_hbm, m_v)
    pltpu.sync_copy(rows_hbm.at[wid], r_v)
    pltpu.sync_copy(cols_hbm.at[wid], c_v)
    z16 = jnp.zeros((16,), jnp.float32)

    def zbody(i, _):
        s_v[pl.ds(i * 16, 16)] = z16
        return 0

    lax.fori_loop(0, NP // 16, zbody, 0)

    def ebody(i, _):
        ridx = r_v[pl.ds(i * 16, 16)]
        cidx = c_v[pl.ds(i * 16, 16)]
        val = plsc.load_gather(m_v, [ridx])
        plsc.addupdate_scatter(s_v, [cidx], val)
        return 0

    lax.fori_loop(0, EPW // 16, ebody, 0)
    pltpu.sync_copy(s_v, out_hbm.at[wid])


@functools.partial(
    pl.kernel,
    mesh=_mesh,
    compiler_params=pltpu.CompilerParams(use_tc_tiling_on_sc=False,
                                         needs_layout_passes=False),
    out_type=jax.ShapeDtypeStruct((2 * NP, HH), jnp.float32),
    scratch_types=[
        pltpu.VMEM((NCT, CH), jnp.int32),
        pltpu.VMEM((NCT, CH), jnp.int32),
        pltpu.VMEM((CH, HH), jnp.float32),
        pltpu.VMEM((CH, HH), jnp.float32),
        pltpu.VMEM((128, HH), jnp.float32),
        pltpu.VMEM_SHARED((NP, HH), jnp.float32),
        pltpu.SemaphoreType.DMA,
        pltpu.SemaphoreType.DMA,
        pltpu.SemaphoreType.DMA,
        pltpu.SemaphoreType.DMA,
    ],
)
def _row_sc(y0_hbm, y1_hbm, rows_hbm, cols_hbm, z_hbm, out_hbm,
            r_v, c_v, buf0, buf1, zbuf, agg_sp, semg0, semg1, sems0, sems1):
    cid = lax.axis_index("c")
    sid = lax.axis_index("s")
    pltpu.sync_copy(z_hbm, zbuf)
    base = sid * RPT
    for t in range(RPT // 128):
        pltpu.sync_copy(zbuf, agg_sp.at[pl.ds(base + t * 128, 128)])
    plsc.subcore_barrier()
    pltpu.sync_copy(rows_hbm.at[sid], r_v)
    pltpu.sync_copy(cols_hbm.at[sid], c_v)

    for c_static, y_hbm in ((0, y0_hbm), (1, y1_hbm)):
        @pl.when(cid == c_static)
        def _():
            pltpu.async_copy(y_hbm.at[r_v.at[0]], buf0, semg0)

            def pair(gp, _):
                g0 = gp * 2
                pltpu.async_copy(y_hbm.at[r_v.at[g0 + 1]], buf1, semg1)
                pltpu.make_async_copy(y_hbm.at[r_v.at[g0]], buf0, semg0).wait()
                s0 = pltpu.async_copy(buf0, agg_sp.at[c_v.at[g0]], sems0,
                                      add=True)
                pltpu.make_async_copy(y_hbm.at[r_v.at[g0 + 1]], buf1,
                                      semg1).wait()
                s1 = pltpu.async_copy(buf1, agg_sp.at[c_v.at[g0 + 1]], sems1,
                                      add=True)
                s0.wait()

                @pl.when(g0 + 2 < NCT)
                def _():
                    pltpu.async_copy(y_hbm.at[r_v.at[g0 + 2]], buf0, semg0)

                s1.wait()
                return 0

            lax.fori_loop(0, NCT // 2, pair, 0)

    plsc.subcore_barrier()
    for t in range(RPT // 128):
        pltpu.sync_copy(agg_sp.at[pl.ds(base + t * 128, 128)], zbuf)
        pltpu.sync_copy(zbuf, out_hbm.at[pl.ds(cid * NP + base + t * 128, 128)])


@functools.partial(
    pl.kernel,
    mesh=_mesh,
    compiler_params=pltpu.CompilerParams(use_tc_tiling_on_sc=False,
                                         needs_layout_passes=False),
    out_type=(jax.ShapeDtypeStruct((16, CAP), jnp.int32),
              jax.ShapeDtypeStruct((16, CAP), jnp.int32),
              jax.ShapeDtypeStruct((16, 16), jnp.int32)),
    scratch_types=[
        pltpu.VMEM((NP,), jnp.float32),
        pltpu.VMEM((EPT,), jnp.int32),
        pltpu.VMEM((EPT,), jnp.int32),
        pltpu.VMEM((CAP,), jnp.int32),
        pltpu.VMEM((CAP,), jnp.int32),
    ],
)
def _compact_sc(rows_hbm, cols_hbm, m_hbm, crows_hbm, ccols_hbm, counts_hbm,
                m_v, r_v, c_v, cr_v, cc_v):
    cid = lax.axis_index("c")
    sid = lax.axis_index("s")

    @pl.when(cid == 0)
    def _():
        pltpu.sync_copy(m_hbm, m_v)
        pltpu.sync_copy(rows_hbm.at[sid], r_v)
        pltpu.sync_copy(cols_hbm.at[sid], c_v)

        def ebody(i, off):
            rv = r_v[pl.ds(i * 16, 16)]
            cv = c_v[pl.ds(i * 16, 16)]
            mr = plsc.load_gather(m_v, [rv])
            mc = plsc.load_gather(m_v, [cv])
            keep = (mr > 0.0) & (mc > 0.0)
            plsc.store_compressed(cr_v.at[pl.ds(off, 16)], rv, mask=keep)
            plsc.store_compressed(cc_v.at[pl.ds(off, 16)], cv, mask=keep)
            nkeep = plsc.all_reduce_population_count(keep)
            return off + jnp.max(nkeep)

        cnt = lax.fori_loop(0, EPT // 16, ebody, jnp.int32(0))
        padv = jnp.full((16,), N, jnp.int32)
        for t in range(16):
            cr_v[pl.ds(cnt + t * 16, 16)] = padv
            cc_v[pl.ds(cnt + t * 16, 16)] = padv
        cntp = ((cnt + PAIR - 1) // PAIR) * PAIR
        pltpu.sync_copy(cr_v, crows_hbm.at[sid])
        pltpu.sync_copy(cc_v, ccols_hbm.at[sid])
        cr_v[pl.ds(CAP - 16, 16)] = jnp.full((16,), cntp, jnp.int32)
        pltpu.sync_copy(cr_v.at[pl.ds(CAP - 16, 16)], counts_hbm.at[sid])


@functools.partial(
    pl.kernel,
    mesh=_mesh,
    compiler_params=pltpu.CompilerParams(use_tc_tiling_on_sc=False,
                                         needs_layout_passes=False),
    out_type=jax.ShapeDtypeStruct((2 * NP, HH), jnp.float32),
    scratch_types=[
        pltpu.VMEM((CAPC, CH), jnp.int32),
        pltpu.VMEM((CAPC, CH), jnp.int32),
        pltpu.VMEM((16,), jnp.int32),
        pltpu.VMEM((CH, HH), jnp.float32),
        pltpu.VMEM((CH, HH), jnp.float32),
        pltpu.VMEM((128, HH), jnp.float32),
        pltpu.VMEM_SHARED((NP, HH), jnp.float32),
        pltpu.SemaphoreType.DMA,
        pltpu.SemaphoreType.DMA,
        pltpu.SemaphoreType.DMA,
        pltpu.SemaphoreType.DMA,
    ],
)
def _row_dyn_sc(y0_hbm, y1_hbm, crows_hbm, ccols_hbm, counts_hbm, z_hbm,
                out_hbm, r_v, c_v, cnt_vv, buf0, buf1, zbuf, agg_sp,
                semg0, semg1, sems0, sems1):
    cid = lax.axis_index("c")
    sid = lax.axis_index("s")
    pltpu.sync_copy(z_hbm, zbuf)
    base = sid * RPT
    for t in range(RPT // 128):
        pltpu.sync_copy(zbuf, agg_sp.at[pl.ds(base + t * 128, 128)])
    plsc.subcore_barrier()
    pltpu.sync_copy(crows_hbm.at[sid], r_v)
    pltpu.sync_copy(ccols_hbm.at[sid], c_v)
    pltpu.sync_copy(counts_hbm.at[sid], cnt_vv)
    nch = jnp.max(cnt_vv[...]) // CH

    for c_static, y_hbm in ((0, y0_hbm), (1, y1_hbm)):
        @pl.when((cid == c_static) & (nch > 0))
        def _():
            pltpu.async_copy(y_hbm.at[r_v.at[0]], buf0, semg0)

            def pair(gp, _):
                g0 = gp * 2
                pltpu.async_copy(y_hbm.at[r_v.at[g0 + 1]], buf1, semg1)
                pltpu.make_async_copy(y_hbm.at[r_v.at[g0]], buf0, semg0).wait()
                s0 = pltpu.async_copy(buf0, agg_sp.at[c_v.at[g0]], sems0,
                                      add=True)
                pltpu.make_async_copy(y_hbm.at[r_v.at[g0 + 1]], buf1,
                                      semg1).wait()
                s1 = pltpu.async_copy(buf1, agg_sp.at[c_v.at[g0 + 1]], sems1,
                                      add=True)
                s0.wait()

                @pl.when(g0 + 2 < nch)
                def _():
                    pltpu.async_copy(y_hbm.at[r_v.at[g0 + 2]], buf0, semg0)

                s1.wait()
                return 0

            lax.fori_loop(0, nch // 2, pair, 0)

    plsc.subcore_barrier()
    for t in range(RPT // 128):
        pltpu.sync_copy(agg_sp.at[pl.ds(base + t * 128, 128)], zbuf)
        pltpu.sync_copy(zbuf, out_hbm.at[pl.ds(cid * NP + base + t * 128, 128)])


def _dot(a, b):
    return jnp.dot(a, b, preferred_element_type=jnp.float32,
                   precision=lax.Precision.HIGHEST)


def _prep_first_body(x_ref, w_ref, sp_ref, m_ref, y_ref, dinv_ref):
    s = jnp.sum(sp_ref[...], axis=1, keepdims=True)
    dinv = m_ref[...] * lax.rsqrt(s + 1.0)
    y_ref[...] = dinv * _dot(x_ref[...], w_ref[...])
    dinv_ref[...] = dinv


def _prep_gated_body(x_ref, g_ref, w_ref, sp_ref, m_ref, y_ref, dinv_ref):
    s = jnp.sum(sp_ref[...], axis=1, keepdims=True)
    dinv = m_ref[...] * lax.rsqrt(s + 1.0)
    y_ref[...] = dinv * _dot(x_ref[...] * g_ref[...], w_ref[...])
    dinv_ref[...] = dinv


def _prep_up_body(res_ref, xd_ref, md_ref, w_ref, dinv_ref, y_ref):
    xin = res_ref[...] + xd_ref[...] * md_ref[...]
    y_ref[...] = dinv_ref[...] * _dot(xin, w_ref[...])


def _combine_body(relu, agg_ref, y_ref, dinv_ref, b_ref, o_ref):
    agg = jnp.concatenate([agg_ref[0:NP, :], agg_ref[NP:2 * NP, :]], axis=1)
    out = dinv_ref[...] * (agg + y_ref[...]) + b_ref[...]
    o_ref[...] = jnp.maximum(out, 0.0) if relu else out


def _score_body(h_ref, p_ref, o_ref):
    nrm = jnp.sqrt(jnp.sum(p_ref[...] * p_ref[...]))
    o_ref[...] = _dot(h_ref[...], p_ref[...]) / nrm


def _combine_score_body(agg_ref, y_ref, dinv_ref, b_ref, p_ref,
                        h_ref, scr_ref):
    agg = jnp.concatenate([agg_ref[0:NP, :], agg_ref[NP:2 * NP, :]], axis=1)
    h = jnp.maximum(dinv_ref[...] * (agg + y_ref[...]) + b_ref[...], 0.0)
    h_ref[...] = h
    nrm = jnp.sqrt(jnp.sum(p_ref[...] * p_ref[...]))
    scr_ref[...] = _dot(h, p_ref[...]) / nrm


def _up_fuse_body(agg0_ref, agg1_ref, y_ref, dinvp_ref, b_ref, res_ref,
                  md_ref, w_ref, dinvt_ref, y2_ref):
    agg = jnp.concatenate([agg0_ref[...], agg1_ref[...]], axis=1)
    h = jnp.maximum(dinvp_ref[...] * (agg + y_ref[...]) + b_ref[...], 0.0)
    xin = res_ref[...] + h * md_ref[...]
    y2_ref[...] = dinvt_ref[...] * _dot(xin, w_ref[...])


_UPBS = 2048


def _up_fuse_call(agg, y, dinvp, b, res, md, w, dinvt):
    nb = NP // _UPBS
    return pl.pallas_call(
        _up_fuse_body,
        grid=(nb,),
        in_specs=[
            pl.BlockSpec((_UPBS, HH), lambda i: (i, 0)),
            pl.BlockSpec((_UPBS, HH), lambda i: (NP // _UPBS + i, 0)),
            pl.BlockSpec((_UPBS, H), lambda i: (i, 0)),
            pl.BlockSpec((_UPBS, 1), lambda i: (i, 0)),
            pl.BlockSpec((1, H), lambda i: (0, 0)),
            pl.BlockSpec((_UPBS, H), lambda i: (i, 0)),
            pl.BlockSpec((_UPBS, 1), lambda i: (i, 0)),
            pl.BlockSpec((H, H), lambda i: (0, 0)),
            pl.BlockSpec((_UPBS, 1), lambda i: (i, 0)),
        ],
        out_specs=pl.BlockSpec((_UPBS, H), lambda i: (i, 0)),
        out_shape=jax.ShapeDtypeStruct((NP, H), jnp.float32),
    )(agg, agg, y, dinvp, b, res, md, w, dinvt)


def _select_body(k, scr_ref, m_ref, sel_ref, gate_ref):
    scr = scr_ref[...]
    bits = lax.bitcast_convert_type(scr, jnp.uint32)
    okey = jnp.where(bits >= jnp.uint32(0x80000000), ~bits,
                     bits | jnp.uint32(0x80000000))
    skey = jnp.where(m_ref[...] > 0, okey, jnp.uint32(0))
    r_i = lax.broadcasted_iota(jnp.uint32, skey.shape, 0)
    c_i = lax.broadcasted_iota(jnp.uint32, skey.shape, 1)
    lo = jnp.uint32(16383) - (r_i * jnp.uint32(128) + c_i)

    def hib(t, carry):
        pref, rem = carry
        sh = 31 - t
        cand = pref | (jnp.uint32(1) << sh)
        c = jnp.sum(((skey >> sh) == (cand >> sh)).astype(jnp.int32))
        take = c >= rem
        return (jnp.where(take, cand, pref), jnp.where(take, rem, rem - c))

    th, rem = lax.fori_loop(0, 32, hib, (jnp.uint32(0), jnp.int32(k)))

    def lob(t, carry):
        plo, rem = carry
        sh = 13 - t
        cand = plo | (jnp.uint32(1) << sh)
        c = jnp.sum(((skey == th) & ((lo >> sh) == (cand >> sh)))
                    .astype(jnp.int32))
        take = c >= rem
        return (jnp.where(take, cand, plo), jnp.where(take, rem, rem - c))

    tl, _ = lax.fori_loop(0, 14, lob, (jnp.uint32(0), rem))
    sel = (skey > th) | ((skey == th) & (lo >= tl))
    self_f = sel.astype(jnp.float32)
    sel_ref[...] = self_f
    gate_ref[...] = jnp.tanh(scr) * self_f


def _tc_call(body, out_shapes, *args):
    return pl.pallas_call(
        body,
        out_shape=out_shapes,
    )(*args)


def _gcn_layer(x, gate, up_pair, w, b, dinv, sparts, m, edges, zrow, relu):
    if up_pair is not None:
        xd, md = up_pair
        y = _tc_call(_prep_up_body,
                     jax.ShapeDtypeStruct((NP, H), jnp.float32),
                     x, xd, md, w, dinv)
    elif gate is None:
        y, dinv = _tc_call(_prep_first_body,
                           (jax.ShapeDtypeStruct((NP, H), jnp.float32),
                            jax.ShapeDtypeStruct((NP, 1), jnp.float32)),
                           x, w, sparts, m)
    else:
        y, dinv = _tc_call(_prep_gated_body,
                           (jax.ShapeDtypeStruct((NP, H), jnp.float32),
                            jax.ShapeDtypeStruct((NP, 1), jnp.float32)),
                           x, gate, w, sparts, m)
    y0 = lax.slice(y, (0, 0), (NP, HH))
    y1 = lax.slice(y, (0, HH), (NP, H))
    if len(edges) == 2:
        agg = _row_sc(y0, y1, edges[0], edges[1], zrow)
    else:
        agg = _row_dyn_sc(y0, y1, edges[0], edges[1], edges[2], zrow)
    out = _tc_call(functools.partial(_combine_body, relu),
                   jax.ShapeDtypeStruct((NP, H), jnp.float32),
                   agg, y, dinv, b)
    return out, dinv


def kernel(x, edge_index, down_w0, down_b0, down_w1, down_b1, down_w2,
           down_b2, down_w3, down_b3, pool_p0, pool_p1, pool_p2,
           up_w0, up_b0, up_w1, up_b1, up_w2, up_b2):
    f32 = jnp.float32
    rows = edge_index[0]
    cols = edge_index[1]
    rows_fl = rows.reshape(NW, EPW)
    cols_fl = cols.reshape(NW, EPW)
    rows_ch = rows.reshape(16, NCT, CH)
    cols_ch = cols.reshape(16, NCT, CH)
    zrow = jnp.zeros((128, HH), f32)

    xp = jnp.pad(x, ((0, NP - N), (0, 0)))
    m0f = jnp.zeros((NP,), f32).at[:N].set(1.0)
    down_w = [down_w0, down_w1, down_w2, down_w3]
    down_b = [down_b0.reshape(1, H), down_b1.reshape(1, H),
              down_b2.reshape(1, H), down_b3.reshape(1, H)]
    up_w = [up_w0, up_w1, up_w2]
    up_b = [up_b0.reshape(1, H), up_b1.reshape(1, H), up_b2.reshape(1, H)]
    pool_p = [pool_p0.reshape(H, 1), pool_p1.reshape(H, 1),
              pool_p2.reshape(H, 1)]

    def deg(mf):
        sparts = _deg_sc(rows_fl, cols_fl, mf)
        return sparts.T

    def row_full(y):
        y0 = lax.slice(y, (0, 0), (NP, HH))
        y1 = lax.slice(y, (0, HH), (NP, H))
        return _row_sc(y0, y1, rows_ch, cols_ch, zrow)

    def row_dyn(y, ce):
        y0 = lax.slice(y, (0, 0), (NP, HH))
        y1 = lax.slice(y, (0, HH), (NP, H))
        return _row_dyn_sc(y0, y1, ce[0], ce[1], ce[2], zrow)

    def select(scr, mf_prev, k):
        sel2, gate2 = _tc_call(
            functools.partial(_select_body, k),
            (jax.ShapeDtypeStruct((NP // 128, 128), f32),
             jax.ShapeDtypeStruct((NP // 128, 128), f32)),
            scr.reshape(NP // 128, 128), mf_prev.reshape(NP // 128, 128))
        return sel2.reshape(NP), gate2.reshape(NP, 1)

    sds = jax.ShapeDtypeStruct
    masks_f = [m0f]
    masks_c = [m0f.reshape(NP, 1)]
    hs = []
    dinvs = []
    gate = None
    comp_edges = None
    h_in = xp
    scr = None
    n_act = N
    ks = []
    for i in range(DEPTH + 1):
        sparts = deg(masks_f[i])
        if i == 0:
            y, dinv = _tc_call(_prep_first_body,
                               (sds((NP, H), f32), sds((NP, 1), f32)),
                               h_in, down_w[i], sparts, masks_c[i])
        else:
            y, dinv = _tc_call(_prep_gated_body,
                               (sds((NP, H), f32), sds((NP, 1), f32)),
                               h_in, gate, down_w[i], sparts, masks_c[i])
        agg = row_full(y) if i == 0 else row_dyn(y, comp_edges)
        dinvs.append(dinv)
        if i == DEPTH:
            agg3, y3, dinv3 = agg, y, dinv
            break
        h, scr = _tc_call(_combine_score_body,
                          (sds((NP, H), f32), sds((NP, 1), f32)),
                          agg, y, dinv, down_b[i], pool_p[i])
        hs.append(h)
        k = (n_act + 1) // 2
        ks.append(k)
        mf, gate = select(scr, masks_f[i], k)
        masks_f.append(mf)
        masks_c.append(mf.reshape(NP, 1))
        n_act = k
        h_in = h
        if i == 0:
            rows_t = rows.reshape(16, EPT)
            cols_t = cols.reshape(16, EPT)
            cr, cc, cnts = _compact_sc(rows_t, cols_t, mf)
            cr3 = lax.slice(cr, (0, 0), (16, CAPC * CH)).reshape(16, CAPC, CH)
            cc3 = lax.slice(cc, (0, 0), (16, CAPC * CH)).reshape(16, CAPC, CH)
            comp_edges = (cr3, cc3, cnts)

    agg_p, y_p = agg3, y3
    for i in range(DEPTH):
        j = DEPTH - 1 - i
        b_p = down_b[DEPTH] if i == 0 else up_b[i - 1]
        dinv_p = dinvs[j + 1]
        y_n = _up_fuse_call(agg_p, y_p, dinv_p, b_p, hs[j], masks_c[j + 1],
                            up_w[i], dinvs[j])
        agg_p = row_full(y_n) if j == 0 else row_dyn(y_n, comp_edges)
        y_p = y_n
    out = _tc_call(functools.partial(_combine_body, False),
                   sds((NP, H), f32), agg_p, y_p, dinvs[0], up_b[DEPTH - 1])
    return out[:N]


# speedup vs baseline: 61.9848x; 1.1480x over previous
import functools

import jax
import jax.numpy as jnp
from jax import lax
from jax.experimental import pallas as pl
from jax.experimental.pallas import tpu as pltpu
from jax.experimental.pallas import tpu_sc as plsc

N = 10000
NP = 10240
E = 320000
H = 128
DEPTH = 3
NW = 32
EPW = E // NW
CH = 125
NCH = EPW // CH
HH = H // 2
EPT = E // 16
NCT = EPT // CH
RPT = NP // 16
CAP = EPT + 256
CAPC = EPT // CH
PAIR = 2 * CH

_mesh = plsc.VectorSubcoreMesh(core_axis_name="c", subcore_axis_name="s")


@functools.partial(
    pl.kernel,
    mesh=_mesh,
    compiler_params=pltpu.CompilerParams(use_tc_tiling_on_sc=False,
                                         needs_layout_passes=False),
    out_type=jax.ShapeDtypeStruct((NW, NP), jnp.float32),
    scratch_types=[
        pltpu.VMEM((NP,), jnp.float32),
        pltpu.VMEM((EPW,), jnp.int32),
        pltpu.VMEM((EPW,), jnp.int32),
        pltpu.VMEM((NP,), jnp.float32),
    ],
)
def _deg_sc(rows_hbm, cols_hbm, m_hbm, out_hbm, m_v, r_v, c_v, s_v):
    cid = lax.axis_index("c")
    sid = lax.axis_index("s")
    wid = cid * 16 + sid
    pltpu.sync_copy(m_hbm, m_v)
    pltpu.sync_copy(rows_hbm.at[wid], r_v)
    pltpu.sync_copy(cols_hbm.at[wid], c_v)
    z16 = jnp.zeros((16,), jnp.float32)

    def zbody(i, _):
        s_v[pl.ds(i * 16, 16)] = z16
        return 0

    lax.fori_loop(0, NP // 16, zbody, 0)

    def ebody(i, _):
        ridx = r_v[pl.ds(i * 16, 16)]
        cidx = c_v[pl.ds(i * 16, 16)]
        val = plsc.load_gather(m_v, [ridx])
        plsc.addupdate_scatter(s_v, [cidx], val)
        return 0

    lax.fori_loop(0, EPW // 16, ebody, 0)
    pltpu.sync_copy(s_v, out_hbm.at[wid])


@functools.partial(
    pl.kernel,
    mesh=_mesh,
    compiler_params=pltpu.CompilerParams(use_tc_tiling_on_sc=False,
                                         needs_layout_passes=False),
    out_type=jax.ShapeDtypeStruct((2 * NP, HH), jnp.float32),
    scratch_types=[
        pltpu.VMEM((NCT, CH), jnp.int32),
        pltpu.VMEM((NCT, CH), jnp.int32),
        pltpu.VMEM((CH, HH), jnp.float32),
        pltpu.VMEM((CH, HH), jnp.float32),
        pltpu.VMEM((128, HH), jnp.float32),
        pltpu.VMEM_SHARED((NP, HH), jnp.float32),
        pltpu.SemaphoreType.DMA,
        pltpu.SemaphoreType.DMA,
        pltpu.SemaphoreType.DMA,
        pltpu.SemaphoreType.DMA,
    ],
)
def _row_sc(y0_hbm, y1_hbm, rows_hbm, cols_hbm, z_hbm, out_hbm,
            r_v, c_v, buf0, buf1, zbuf, agg_sp, semg0, semg1, sems0, sems1):
    cid = lax.axis_index("c")
    sid = lax.axis_index("s")
    pltpu.sync_copy(z_hbm, zbuf)
    base = sid * RPT
    for t in range(RPT // 128):
        pltpu.sync_copy(zbuf, agg_sp.at[pl.ds(base + t * 128, 128)])
    plsc.subcore_barrier()
    pltpu.sync_copy(rows_hbm.at[sid], r_v)
    pltpu.sync_copy(cols_hbm.at[sid], c_v)

    for c_static, y_hbm in ((0, y0_hbm), (1, y1_hbm)):
        @pl.when(cid == c_static)
        def _():
            pltpu.async_copy(y_hbm.at[r_v.at[0]], buf0, semg0)

            def pair(gp, _):
                g0 = gp * 2
                pltpu.async_copy(y_hbm.at[r_v.at[g0 + 1]], buf1, semg1)
                pltpu.make_async_copy(y_hbm.at[r_v.at[g0]], buf0, semg0).wait()
                s0 = pltpu.async_copy(buf0, agg_sp.at[c_v.at[g0]], sems0,
                                      add=True)
                pltpu.make_async_copy(y_hbm.at[r_v.at[g0 + 1]], buf1,
                                      semg1).wait()
                s1 = pltpu.async_copy(buf1, agg_sp.at[c_v.at[g0 + 1]], sems1,
                                      add=True)
                s0.wait()

                @pl.when(g0 + 2 < NCT)
                def _():
                    pltpu.async_copy(y_hbm.at[r_v.at[g0 + 2]], buf0, semg0)

                s1.wait()
                return 0

            lax.fori_loop(0, NCT // 2, pair, 0)

    plsc.subcore_barrier()
    for t in range(RPT // 128):
        pltpu.sync_copy(agg_sp.at[pl.ds(base + t * 128, 128)], zbuf)
        pltpu.sync_copy(zbuf, out_hbm.at[pl.ds(cid * NP + base + t * 128, 128)])


@functools.partial(
    pl.kernel,
    mesh=_mesh,
    compiler_params=pltpu.CompilerParams(use_tc_tiling_on_sc=False,
                                         needs_layout_passes=False),
    out_type=(jax.ShapeDtypeStruct((16, CAP), jnp.int32),
              jax.ShapeDtypeStruct((16, CAP), jnp.int32),
              jax.ShapeDtypeStruct((16, 16), jnp.int32)),
    scratch_types=[
        pltpu.VMEM((NP,), jnp.float32),
        pltpu.VMEM((EPT,), jnp.int32),
        pltpu.VMEM((EPT,), jnp.int32),
        pltpu.VMEM((CAP,), jnp.int32),
        pltpu.VMEM((CAP,), jnp.int32),
    ],
)
def _compact_sc(rows_hbm, cols_hbm, m_hbm, crows_hbm, ccols_hbm, counts_hbm,
                m_v, r_v, c_v, cr_v, cc_v):
    cid = lax.axis_index("c")
    sid = lax.axis_index("s")

    @pl.when(cid == 0)
    def _():
        pltpu.sync_copy(m_hbm, m_v)
        pltpu.sync_copy(rows_hbm.at[sid], r_v)
        pltpu.sync_copy(cols_hbm.at[sid], c_v)

        def ebody(i, off):
            rv = r_v[pl.ds(i * 16, 16)]
            cv = c_v[pl.ds(i * 16, 16)]
            mr = plsc.load_gather(m_v, [rv])
            mc = plsc.load_gather(m_v, [cv])
            keep = (mr > 0.0) & (mc > 0.0)
            plsc.store_compressed(cr_v.at[pl.ds(off, 16)], rv, mask=keep)
            plsc.store_compressed(cc_v.at[pl.ds(off, 16)], cv, mask=keep)
            nkeep = plsc.all_reduce_population_count(keep)
            return off + jnp.max(nkeep)

        cnt = lax.fori_loop(0, EPT // 16, ebody, jnp.int32(0))
        padv = jnp.full((16,), N, jnp.int32)
        for t in range(16):
            cr_v[pl.ds(cnt + t * 16, 16)] = padv
            cc_v[pl.ds(cnt + t * 16, 16)] = padv
        cntp = ((cnt + PAIR - 1) // PAIR) * PAIR
        pltpu.sync_copy(cr_v, crows_hbm.at[sid])
        pltpu.sync_copy(cc_v, ccols_hbm.at[sid])
        cr_v[pl.ds(CAP - 16, 16)] = jnp.full((16,), cntp, jnp.int32)
        pltpu.sync_copy(cr_v.at[pl.ds(CAP - 16, 16)], counts_hbm.at[sid])


@functools.partial(
    pl.kernel,
    mesh=_mesh,
    compiler_params=pltpu.CompilerParams(use_tc_tiling_on_sc=False,
                                         needs_layout_passes=False),
    out_type=jax.ShapeDtypeStruct((2 * NP, HH), jnp.float32),
    scratch_types=[
        pltpu.VMEM((CAPC, CH), jnp.int32),
        pltpu.VMEM((CAPC, CH), jnp.int32),
        pltpu.VMEM((16,), jnp.int32),
        pltpu.VMEM((CH, HH), jnp.float32),
        pltpu.VMEM((CH, HH), jnp.float32),
        pltpu.VMEM((128, HH), jnp.float32),
        pltpu.VMEM_SHARED((NP, HH), jnp.float32),
        pltpu.SemaphoreType.DMA,
        pltpu.SemaphoreType.DMA,
        pltpu.SemaphoreType.DMA,
        pltpu.SemaphoreType.DMA,
    ],
)
def _row_dyn_sc(y0_hbm, y1_hbm, crows_hbm, ccols_hbm, counts_hbm, z_hbm,
                out_hbm, r_v, c_v, cnt_vv, buf0, buf1, zbuf, agg_sp,
                semg0, semg1, sems0, sems1):
    cid = lax.axis_index("c")
    sid = lax.axis_index("s")
    pltpu.sync_copy(z_hbm, zbuf)
    base = sid * RPT
    for t in range(RPT // 128):
        pltpu.sync_copy(zbuf, agg_sp.at[pl.ds(base + t * 128, 128)])
    plsc.subcore_barrier()
    pltpu.sync_copy(crows_hbm.at[sid], r_v)
    pltpu.sync_copy(ccols_hbm.at[sid], c_v)
    pltpu.sync_copy(counts_hbm.at[sid], cnt_vv)
    nch = jnp.max(cnt_vv[...]) // CH

    for c_static, y_hbm in ((0, y0_hbm), (1, y1_hbm)):
        @pl.when((cid == c_static) & (nch > 0))
        def _():
            pltpu.async_copy(y_hbm.at[r_v.at[0]], buf0, semg0)

            def pair(gp, _):
                g0 = gp * 2
                pltpu.async_copy(y_hbm.at[r_v.at[g0 + 1]], buf1, semg1)
                pltpu.make_async_copy(y_hbm.at[r_v.at[g0]], buf0, semg0).wait()
                s0 = pltpu.async_copy(buf0, agg_sp.at[c_v.at[g0]], sems0,
                                      add=True)
                pltpu.make_async_copy(y_hbm.at[r_v.at[g0 + 1]], buf1,
                                      semg1).wait()
                s1 = pltpu.async_copy(buf1, agg_sp.at[c_v.at[g0 + 1]], sems1,
                                      add=True)
                s0.wait()

                @pl.when(g0 + 2 < nch)
                def _():
                    pltpu.async_copy(y_hbm.at[r_v.at[g0 + 2]], buf0, semg0)

                s1.wait()
                return 0

            lax.fori_loop(0, nch // 2, pair, 0)

    plsc.subcore_barrier()
    for t in range(RPT // 128):
        pltpu.sync_copy(agg_sp.at[pl.ds(base + t * 128, 128)], zbuf)
        pltpu.sync_copy(zbuf, out_hbm.at[pl.ds(cid * NP + base + t * 128, 128)])


def _dot(a, b):
    return jnp.dot(a, b, preferred_element_type=jnp.float32,
                   precision=lax.Precision.HIGHEST)


def _prep_first_body(x_ref, w_ref, sp_ref, m_ref, y_ref, dinv_ref):
    s = jnp.sum(sp_ref[...], axis=1, keepdims=True)
    dinv = m_ref[...] * lax.rsqrt(s + 1.0)
    y_ref[...] = dinv * _dot(x_ref[...], w_ref[...])
    dinv_ref[...] = dinv


def _prep_gated_body(x_ref, g_ref, w_ref, sp_ref, m_ref, y_ref, dinv_ref):
    s = jnp.sum(sp_ref[...], axis=1, keepdims=True)
    dinv = m_ref[...] * lax.rsqrt(s + 1.0)
    y_ref[...] = dinv * _dot(x_ref[...] * g_ref[...], w_ref[...])
    dinv_ref[...] = dinv


def _prep_up_body(res_ref, xd_ref, md_ref, w_ref, dinv_ref, y_ref):
    xin = res_ref[...] + xd_ref[...] * md_ref[...]
    y_ref[...] = dinv_ref[...] * _dot(xin, w_ref[...])


def _combine_body(relu, agg_ref, y_ref, dinv_ref, b_ref, o_ref):
    agg = jnp.concatenate([agg_ref[0:NP, :], agg_ref[NP:2 * NP, :]], axis=1)
    out = dinv_ref[...] * (agg + y_ref[...]) + b_ref[...]
    o_ref[...] = jnp.maximum(out, 0.0) if relu else out


def _score_body(h_ref, p_ref, o_ref):
    nrm = jnp.sqrt(jnp.sum(p_ref[...] * p_ref[...]))
    o_ref[...] = _dot(h_ref[...], p_ref[...]) / nrm


def _combine_score_body(agg_ref, y_ref, dinv_ref, b_ref, p_ref,
                        h_ref, scr_ref):
    agg = jnp.concatenate([agg_ref[0:NP, :], agg_ref[NP:2 * NP, :]], axis=1)
    h = jnp.maximum(dinv_ref[...] * (agg + y_ref[...]) + b_ref[...], 0.0)
    h_ref[...] = h
    nrm = jnp.sqrt(jnp.sum(p_ref[...] * p_ref[...]))
    scr_ref[...] = _dot(h, p_ref[...]) / nrm


def _up_fuse_body(agg0_ref, agg1_ref, y_ref, dinvp_ref, b_ref, res_ref,
                  md_ref, w_ref, dinvt_ref, y2_ref):
    agg = jnp.concatenate([agg0_ref[...], agg1_ref[...]], axis=1)
    h = jnp.maximum(dinvp_ref[...] * (agg + y_ref[...]) + b_ref[...], 0.0)
    xin = res_ref[...] + h * md_ref[...]
    y2_ref[...] = dinvt_ref[...] * _dot(xin, w_ref[...])


_UPBS = 2048


def _up_fuse_call(agg, y, dinvp, b, res, md, w, dinvt):
    nb = NP // _UPBS
    return pl.pallas_call(
        _up_fuse_body,
        grid=(nb,),
        in_specs=[
            pl.BlockSpec((_UPBS, HH), lambda i: (i, 0)),
            pl.BlockSpec((_UPBS, HH), lambda i: (NP // _UPBS + i, 0)),
            pl.BlockSpec((_UPBS, H), lambda i: (i, 0)),
            pl.BlockSpec((_UPBS, 1), lambda i: (i, 0)),
            pl.BlockSpec((1, H), lambda i: (0, 0)),
            pl.BlockSpec((_UPBS, H), lambda i: (i, 0)),
            pl.BlockSpec((_UPBS, 1), lambda i: (i, 0)),
            pl.BlockSpec((H, H), lambda i: (0, 0)),
            pl.BlockSpec((_UPBS, 1), lambda i: (i, 0)),
        ],
        out_specs=pl.BlockSpec((_UPBS, H), lambda i: (i, 0)),
        out_shape=jax.ShapeDtypeStruct((NP, H), jnp.float32),
    )(agg, agg, y, dinvp, b, res, md, w, dinvt)


def _select_body(k, scr_ref, m_ref, sel_ref, gate_ref):
    scr = scr_ref[...]
    bits = lax.bitcast_convert_type(scr, jnp.uint32)
    okey = jnp.where(bits >= jnp.uint32(0x80000000), ~bits,
                     bits | jnp.uint32(0x80000000))
    skey = jnp.where(m_ref[...] > 0, okey, jnp.uint32(0))
    r_i = lax.broadcasted_iota(jnp.uint32, skey.shape, 0)
    c_i = lax.broadcasted_iota(jnp.uint32, skey.shape, 1)
    lo = jnp.uint32(16383) - (r_i * jnp.uint32(128) + c_i)

    def hib(t, carry):
        pref, rem = carry
        sh = 31 - t
        cand = pref | (jnp.uint32(1) << sh)
        c = jnp.sum(((skey >> sh) == (cand >> sh)).astype(jnp.int32))
        take = c >= rem
        return (jnp.where(take, cand, pref), jnp.where(take, rem, rem - c))

    th, rem = lax.fori_loop(0, 32, hib, (jnp.uint32(0), jnp.int32(k)))

    def lob(t, carry):
        plo, rem = carry
        sh = 13 - t
        cand = plo | (jnp.uint32(1) << sh)
        c = jnp.sum(((skey == th) & ((lo >> sh) == (cand >> sh)))
                    .astype(jnp.int32))
        take = c >= rem
        return (jnp.where(take, cand, plo), jnp.where(take, rem, rem - c))

    tl, _ = lax.fori_loop(0, 14, lob, (jnp.uint32(0), rem))
    sel = (skey > th) | ((skey == th) & (lo >= tl))
    self_f = sel.astype(jnp.float32)
    sel_ref[...] = self_f
    gate_ref[...] = jnp.tanh(scr) * self_f


def _tc_call(body, out_shapes, *args):
    return pl.pallas_call(
        body,
        out_shape=out_shapes,
    )(*args)


def _gcn_layer(x, gate, up_pair, w, b, dinv, sparts, m, edges, zrow, relu):
    if up_pair is not None:
        xd, md = up_pair
        y = _tc_call(_prep_up_body,
                     jax.ShapeDtypeStruct((NP, H), jnp.float32),
                     x, xd, md, w, dinv)
    elif gate is None:
        y, dinv = _tc_call(_prep_first_body,
                           (jax.ShapeDtypeStruct((NP, H), jnp.float32),
                            jax.ShapeDtypeStruct((NP, 1), jnp.float32)),
                           x, w, sparts, m)
    else:
        y, dinv = _tc_call(_prep_gated_body,
                           (jax.ShapeDtypeStruct((NP, H), jnp.float32),
                            jax.ShapeDtypeStruct((NP, 1), jnp.float32)),
                           x, gate, w, sparts, m)
    y0 = lax.slice(y, (0, 0), (NP, HH))
    y1 = lax.slice(y, (0, HH), (NP, H))
    if len(edges) == 2:
        agg = _row_sc(y0, y1, edges[0], edges[1], zrow)
    else:
        agg = _row_dyn_sc(y0, y1, edges[0], edges[1], edges[2], zrow)
    out = _tc_call(functools.partial(_combine_body, relu),
                   jax.ShapeDtypeStruct((NP, H), jnp.float32),
                   agg, y, dinv, b)
    return out, dinv


def kernel(x, edge_index, down_w0, down_b0, down_w1, down_b1, down_w2,
           down_b2, down_w3, down_b3, pool_p0, pool_p1, pool_p2,
           up_w0, up_b0, up_w1, up_b1, up_w2, up_b2):
    f32 = jnp.float32
    rows = edge_index[0]
    cols = edge_index[1]
    rows_fl = rows.reshape(NW, EPW)
    cols_fl = cols.reshape(NW, EPW)
    rows_ch = rows.reshape(16, NCT, CH)
    cols_ch = cols.reshape(16, NCT, CH)
    zrow = jnp.zeros((128, HH), f32)

    xp = jnp.pad(x, ((0, NP - N), (0, 0)))
    m0f = jnp.zeros((NP,), f32).at[:N].set(1.0)
    down_w = [down_w0, down_w1, down_w2, down_w3]
    down_b = [down_b0.reshape(1, H), down_b1.reshape(1, H),
              down_b2.reshape(1, H), down_b3.reshape(1, H)]
    up_w = [up_w0, up_w1, up_w2]
    up_b = [up_b0.reshape(1, H), up_b1.reshape(1, H), up_b2.reshape(1, H)]
    pool_p = [pool_p0.reshape(H, 1), pool_p1.reshape(H, 1),
              pool_p2.reshape(H, 1)]

    def deg(mf):
        sparts = _deg_sc(rows_fl, cols_fl, mf)
        return sparts.T

    def row_full(y):
        y0 = lax.slice(y, (0, 0), (NP, HH))
        y1 = lax.slice(y, (0, HH), (NP, H))
        return _row_sc(y0, y1, rows_ch, cols_ch, zrow)

    def row_dyn(y, ce):
        y0 = lax.slice(y, (0, 0), (NP, HH))
        y1 = lax.slice(y, (0, HH), (NP, H))
        return _row_dyn_sc(y0, y1, ce[0], ce[1], ce[2], zrow)

    def select(scr, mf_prev, k):
        sel2, gate2 = _tc_call(
            functools.partial(_select_body, k),
            (jax.ShapeDtypeStruct((NP // 128, 128), f32),
             jax.ShapeDtypeStruct((NP // 128, 128), f32)),
            scr.reshape(NP // 128, 128), mf_prev.reshape(NP // 128, 128))
        return sel2.reshape(NP), gate2.reshape(NP, 1)

    sds = jax.ShapeDtypeStruct
    def compact(mf):
        rows_t = rows.reshape(16, EPT)
        cols_t = cols.reshape(16, EPT)
        cr, cc, cnts = _compact_sc(rows_t, cols_t, mf)
        cr3 = lax.slice(cr, (0, 0), (16, CAPC * CH)).reshape(16, CAPC, CH)
        cc3 = lax.slice(cc, (0, 0), (16, CAPC * CH)).reshape(16, CAPC, CH)
        return (cr3, cc3, cnts)

    masks_f = [m0f]
    masks_c = [m0f.reshape(NP, 1)]
    hs = []
    dinvs = []
    gate = None
    comp = [None]
    h_in = xp
    scr = None
    n_act = N
    for i in range(DEPTH + 1):
        sparts = deg(masks_f[i])
        if i == 0:
            y, dinv = _tc_call(_prep_first_body,
                               (sds((NP, H), f32), sds((NP, 1), f32)),
                               h_in, down_w[i], sparts, masks_c[i])
        else:
            y, dinv = _tc_call(_prep_gated_body,
                               (sds((NP, H), f32), sds((NP, 1), f32)),
                               h_in, gate, down_w[i], sparts, masks_c[i])
        agg = row_full(y) if i == 0 else row_dyn(y, comp[i])
        dinvs.append(dinv)
        if i == DEPTH:
            agg3, y3 = agg, y
            break
        h, scr = _tc_call(_combine_score_body,
                          (sds((NP, H), f32), sds((NP, 1), f32)),
                          agg, y, dinv, down_b[i], pool_p[i])
        hs.append(h)
        k = (n_act + 1) // 2
        mf, gate = select(scr, masks_f[i], k)
        masks_f.append(mf)
        masks_c.append(mf.reshape(NP, 1))
        n_act = k
        h_in = h
        comp.append(compact(mf))

    agg_p, y_p = agg3, y3
    for i in range(DEPTH):
        j = DEPTH - 1 - i
        b_p = down_b[DEPTH] if i == 0 else up_b[i - 1]
        dinv_p = dinvs[j + 1]
        y_n = _up_fuse_call(agg_p, y_p, dinv_p, b_p, hs[j], masks_c[j + 1],
                            up_w[i], dinvs[j])
        agg_p = row_full(y_n) if j == 0 else row_dyn(y_n, comp[j])
        y_p = y_n
    out = _tc_call(functools.partial(_combine_body, False),
                   sds((NP, H), f32), agg_p, y_p, dinvs[0], up_b[DEPTH - 1])
    return out[:N]


# speedup vs baseline: 62.0267x; 1.0007x over previous
import functools

import jax
import jax.numpy as jnp
from jax import lax
from jax.experimental import pallas as pl
from jax.experimental.pallas import tpu as pltpu
from jax.experimental.pallas import tpu_sc as plsc

N = 10000
NP = 10240
E = 320000
H = 128
DEPTH = 3
NW = 32
EPW = E // NW
CH = 125
NCH = EPW // CH
HH = H // 2
EPT = E // 16
NCT = EPT // CH
RPT = NP // 16
CAP = EPT + 256
CAPC = EPT // CH
PAIR = 2 * CH

_mesh = plsc.VectorSubcoreMesh(core_axis_name="c", subcore_axis_name="s")


@functools.partial(
    pl.kernel,
    mesh=_mesh,
    compiler_params=pltpu.CompilerParams(use_tc_tiling_on_sc=False,
                                         needs_layout_passes=False),
    out_type=jax.ShapeDtypeStruct((NW, NP), jnp.float32),
    scratch_types=[
        pltpu.VMEM((NP,), jnp.float32),
        pltpu.VMEM((EPW,), jnp.int32),
        pltpu.VMEM((EPW,), jnp.int32),
        pltpu.VMEM((NP,), jnp.float32),
    ],
)
def _deg_sc(rows_hbm, cols_hbm, m_hbm, out_hbm, m_v, r_v, c_v, s_v):
    cid = lax.axis_index("c")
    sid = lax.axis_index("s")
    wid = cid * 16 + sid
    pltpu.sync_copy(m_hbm, m_v)
    pltpu.sync_copy(rows_hbm.at[wid], r_v)
    pltpu.sync_copy(cols_hbm.at[wid], c_v)
    z16 = jnp.zeros((16,), jnp.float32)

    def zbody(i, _):
        s_v[pl.ds(i * 16, 16)] = z16
        return 0

    lax.fori_loop(0, NP // 16, zbody, 0)

    def ebody(i, _):
        ridx = r_v[pl.ds(i * 16, 16)]
        cidx = c_v[pl.ds(i * 16, 16)]
        val = plsc.load_gather(m_v, [ridx])
        plsc.addupdate_scatter(s_v, [cidx], val)
        return 0

    lax.fori_loop(0, EPW // 16, ebody, 0)
    pltpu.sync_copy(s_v, out_hbm.at[wid])


@functools.partial(
    pl.kernel,
    mesh=_mesh,
    compiler_params=pltpu.CompilerParams(use_tc_tiling_on_sc=False,
                                         needs_layout_passes=False),
    out_type=jax.ShapeDtypeStruct((2 * NP, HH), jnp.float32),
    scratch_types=[
        pltpu.VMEM((NCT, CH), jnp.int32),
        pltpu.VMEM((NCT, CH), jnp.int32),
        pltpu.VMEM((CH, HH), jnp.float32),
        pltpu.VMEM((CH, HH), jnp.float32),
        pltpu.VMEM((128, HH), jnp.float32),
        pltpu.VMEM_SHARED((NP, HH), jnp.float32),
        pltpu.SemaphoreType.DMA,
        pltpu.SemaphoreType.DMA,
        pltpu.SemaphoreType.DMA,
        pltpu.SemaphoreType.DMA,
    ],
)
def _row_sc(y0_hbm, y1_hbm, rows_hbm, cols_hbm, z_hbm, out_hbm,
            r_v, c_v, buf0, buf1, zbuf, agg_sp, semg0, semg1, sems0, sems1):
    cid = lax.axis_index("c")
    sid = lax.axis_index("s")
    pltpu.sync_copy(z_hbm, zbuf)
    base = sid * RPT
    for t in range(RPT // 128):
        pltpu.sync_copy(zbuf, agg_sp.at[pl.ds(base + t * 128, 128)])
    plsc.subcore_barrier()
    pltpu.sync_copy(rows_hbm.at[sid], r_v)
    pltpu.sync_copy(cols_hbm.at[sid], c_v)

    for c_static, y_hbm in ((0, y0_hbm), (1, y1_hbm)):
        @pl.when(cid == c_static)
        def _():
            pltpu.async_copy(y_hbm.at[r_v.at[0]], buf0, semg0)

            def pair(gp, _):
                g0 = gp * 2
                pltpu.async_copy(y_hbm.at[r_v.at[g0 + 1]], buf1, semg1)
                pltpu.make_async_copy(y_hbm.at[r_v.at[g0]], buf0, semg0).wait()
                s0 = pltpu.async_copy(buf0, agg_sp.at[c_v.at[g0]], sems0,
                                      add=True)
                pltpu.make_async_copy(y_hbm.at[r_v.at[g0 + 1]], buf1,
                                      semg1).wait()
                s1 = pltpu.async_copy(buf1, agg_sp.at[c_v.at[g0 + 1]], sems1,
                                      add=True)
                s0.wait()

                @pl.when(g0 + 2 < NCT)
                def _():
                    pltpu.async_copy(y_hbm.at[r_v.at[g0 + 2]], buf0, semg0)

                s1.wait()
                return 0

            lax.fori_loop(0, NCT // 2, pair, 0)

    plsc.subcore_barrier()
    for t in range(RPT // 128):
        pltpu.sync_copy(agg_sp.at[pl.ds(base + t * 128, 128)], zbuf)
        pltpu.sync_copy(zbuf, out_hbm.at[pl.ds(cid * NP + base + t * 128, 128)])


@functools.partial(
    pl.kernel,
    mesh=_mesh,
    compiler_params=pltpu.CompilerParams(use_tc_tiling_on_sc=False,
                                         needs_layout_passes=False),
    out_type=(jax.ShapeDtypeStruct((16, CAP), jnp.int32),
              jax.ShapeDtypeStruct((16, CAP), jnp.int32),
              jax.ShapeDtypeStruct((16, 16), jnp.int32)),
    scratch_types=[
        pltpu.VMEM((NP,), jnp.float32),
        pltpu.VMEM((EPT,), jnp.int32),
        pltpu.VMEM((EPT,), jnp.int32),
        pltpu.VMEM((CAP,), jnp.int32),
        pltpu.VMEM((CAP,), jnp.int32),
    ],
)
def _compact_sc(rows_hbm, cols_hbm, m_hbm, crows_hbm, ccols_hbm, counts_hbm,
                m_v, r_v, c_v, cr_v, cc_v):
    cid = lax.axis_index("c")
    sid = lax.axis_index("s")

    @pl.when(cid == 0)
    def _():
        pltpu.sync_copy(m_hbm, m_v)
        pltpu.sync_copy(rows_hbm.at[sid], r_v)
        pltpu.sync_copy(cols_hbm.at[sid], c_v)

        def ebody(i, off):
            rv = r_v[pl.ds(i * 16, 16)]
            cv = c_v[pl.ds(i * 16, 16)]
            mr = plsc.load_gather(m_v, [rv])
            mc = plsc.load_gather(m_v, [cv])
            keep = (mr > 0.0) & (mc > 0.0)
            plsc.store_compressed(cr_v.at[pl.ds(off, 16)], rv, mask=keep)
            plsc.store_compressed(cc_v.at[pl.ds(off, 16)], cv, mask=keep)
            nkeep = plsc.all_reduce_population_count(keep)
            return off + jnp.max(nkeep)

        cnt = lax.fori_loop(0, EPT // 16, ebody, jnp.int32(0))
        padv = jnp.full((16,), N, jnp.int32)
        for t in range(16):
            cr_v[pl.ds(cnt + t * 16, 16)] = padv
            cc_v[pl.ds(cnt + t * 16, 16)] = padv
        cntp = ((cnt + PAIR - 1) // PAIR) * PAIR
        pltpu.sync_copy(cr_v, crows_hbm.at[sid])
        pltpu.sync_copy(cc_v, ccols_hbm.at[sid])
        cr_v[pl.ds(CAP - 16, 16)] = jnp.full((16,), cntp, jnp.int32)
        pltpu.sync_copy(cr_v.at[pl.ds(CAP - 16, 16)], counts_hbm.at[sid])


@functools.partial(
    pl.kernel,
    mesh=_mesh,
    compiler_params=pltpu.CompilerParams(use_tc_tiling_on_sc=False,
                                         needs_layout_passes=False),
    out_type=jax.ShapeDtypeStruct((2 * NP, HH), jnp.float32),
    scratch_types=[
        pltpu.VMEM((CAPC, CH), jnp.int32),
        pltpu.VMEM((CAPC, CH), jnp.int32),
        pltpu.VMEM((16,), jnp.int32),
        pltpu.VMEM((CH, HH), jnp.float32),
        pltpu.VMEM((CH, HH), jnp.float32),
        pltpu.VMEM((128, HH), jnp.float32),
        pltpu.VMEM_SHARED((NP, HH), jnp.float32),
        pltpu.SemaphoreType.DMA,
        pltpu.SemaphoreType.DMA,
        pltpu.SemaphoreType.DMA,
        pltpu.SemaphoreType.DMA,
    ],
)
def _row_dyn_sc(y0_hbm, y1_hbm, crows_hbm, ccols_hbm, counts_hbm, z_hbm,
                out_hbm, r_v, c_v, cnt_vv, buf0, buf1, zbuf, agg_sp,
                semg0, semg1, sems0, sems1):
    cid = lax.axis_index("c")
    sid = lax.axis_index("s")
    pltpu.sync_copy(z_hbm, zbuf)
    base = sid * RPT
    for t in range(RPT // 128):
        pltpu.sync_copy(zbuf, agg_sp.at[pl.ds(base + t * 128, 128)])
    plsc.subcore_barrier()
    pltpu.sync_copy(crows_hbm.at[sid], r_v)
    pltpu.sync_copy(ccols_hbm.at[sid], c_v)
    pltpu.sync_copy(counts_hbm.at[sid], cnt_vv)
    nch = jnp.max(cnt_vv[...]) // CH

    for c_static, y_hbm in ((0, y0_hbm), (1, y1_hbm)):
        @pl.when((cid == c_static) & (nch > 0))
        def _():
            pltpu.async_copy(y_hbm.at[r_v.at[0]], buf0, semg0)

            def pair(gp, _):
                g0 = gp * 2
                pltpu.async_copy(y_hbm.at[r_v.at[g0 + 1]], buf1, semg1)
                pltpu.make_async_copy(y_hbm.at[r_v.at[g0]], buf0, semg0).wait()
                s0 = pltpu.async_copy(buf0, agg_sp.at[c_v.at[g0]], sems0,
                                      add=True)
                pltpu.make_async_copy(y_hbm.at[r_v.at[g0 + 1]], buf1,
                                      semg1).wait()
                s1 = pltpu.async_copy(buf1, agg_sp.at[c_v.at[g0 + 1]], sems1,
                                      add=True)
                s0.wait()

                @pl.when(g0 + 2 < nch)
                def _():
                    pltpu.async_copy(y_hbm.at[r_v.at[g0 + 2]], buf0, semg0)

                s1.wait()
                return 0

            lax.fori_loop(0, nch // 2, pair, 0)

    plsc.subcore_barrier()
    for t in range(RPT // 128):
        pltpu.sync_copy(agg_sp.at[pl.ds(base + t * 128, 128)], zbuf)
        pltpu.sync_copy(zbuf, out_hbm.at[pl.ds(cid * NP + base + t * 128, 128)])


def _dot(a, b):
    return jnp.dot(a, b, preferred_element_type=jnp.float32,
                   precision=lax.Precision.HIGHEST)


def _prep_first_body(x_ref, w_ref, sp_ref, m_ref, y_ref, dinv_ref):
    s = jnp.sum(sp_ref[...], axis=1, keepdims=True)
    dinv = m_ref[...] * lax.rsqrt(s + 1.0)
    y_ref[...] = dinv * _dot(x_ref[...], w_ref[...])
    dinv_ref[...] = dinv


def _prep_gated_body(x_ref, g_ref, w_ref, sp_ref, m_ref, y_ref, dinv_ref):
    s = jnp.sum(sp_ref[...], axis=1, keepdims=True)
    dinv = m_ref[...] * lax.rsqrt(s + 1.0)
    y_ref[...] = dinv * _dot(x_ref[...] * g_ref[...], w_ref[...])
    dinv_ref[...] = dinv


def _combine_body(relu, agg_ref, y_ref, dinv_ref, b_ref, o_ref):
    agg = jnp.concatenate([agg_ref[0:NP, :], agg_ref[NP:2 * NP, :]], axis=1)
    out = dinv_ref[...] * (agg + y_ref[...]) + b_ref[...]
    o_ref[...] = jnp.maximum(out, 0.0) if relu else out


def _combine_score_body(agg_ref, y_ref, dinv_ref, b_ref, p_ref,
                        h_ref, scr_ref):
    agg = jnp.concatenate([agg_ref[0:NP, :], agg_ref[NP:2 * NP, :]], axis=1)
    h = jnp.maximum(dinv_ref[...] * (agg + y_ref[...]) + b_ref[...], 0.0)
    h_ref[...] = h
    nrm = jnp.sqrt(jnp.sum(p_ref[...] * p_ref[...]))
    scr_ref[...] = _dot(h, p_ref[...]) / nrm


def _up_fuse_body(agg0_ref, agg1_ref, y_ref, dinvp_ref, b_ref, res_ref,
                  md_ref, w_ref, dinvt_ref, y2_ref):
    agg = jnp.concatenate([agg0_ref[...], agg1_ref[...]], axis=1)
    h = jnp.maximum(dinvp_ref[...] * (agg + y_ref[...]) + b_ref[...], 0.0)
    xin = res_ref[...] + h * md_ref[...]
    y2_ref[...] = dinvt_ref[...] * _dot(xin, w_ref[...])


_UPBS = 2048


def _up_fuse_call(agg, y, dinvp, b, res, md, w, dinvt):
    nb = NP // _UPBS
    return pl.pallas_call(
        _up_fuse_body,
        grid=(nb,),
        in_specs=[
            pl.BlockSpec((_UPBS, HH), lambda i: (i, 0)),
            pl.BlockSpec((_UPBS, HH), lambda i: (NP // _UPBS + i, 0)),
            pl.BlockSpec((_UPBS, H), lambda i: (i, 0)),
            pl.BlockSpec((_UPBS, 1), lambda i: (i, 0)),
            pl.BlockSpec((1, H), lambda i: (0, 0)),
            pl.BlockSpec((_UPBS, H), lambda i: (i, 0)),
            pl.BlockSpec((_UPBS, 1), lambda i: (i, 0)),
            pl.BlockSpec((H, H), lambda i: (0, 0)),
            pl.BlockSpec((_UPBS, 1), lambda i: (i, 0)),
        ],
        out_specs=pl.BlockSpec((_UPBS, H), lambda i: (i, 0)),
        out_shape=jax.ShapeDtypeStruct((NP, H), jnp.float32),
    )(agg, agg, y, dinvp, b, res, md, w, dinvt)


def _select_body(k, scr_ref, m_ref, sel_ref, gate_ref):
    scr = scr_ref[...]
    bits = lax.bitcast_convert_type(scr, jnp.uint32)
    okey = jnp.where(bits >= jnp.uint32(0x80000000), ~bits,
                     bits | jnp.uint32(0x80000000))
    skey = jnp.where(m_ref[...] > 0, okey, jnp.uint32(0))
    r_i = lax.broadcasted_iota(jnp.uint32, skey.shape, 0)
    c_i = lax.broadcasted_iota(jnp.uint32, skey.shape, 1)
    lo = jnp.uint32(16383) - (r_i * jnp.uint32(128) + c_i)

    def hib(t, carry):
        pref, rem = carry
        sh = 31 - t
        cand = pref | (jnp.uint32(1) << sh)
        c = jnp.sum(((skey >> sh) == (cand >> sh)).astype(jnp.int32))
        take = c >= rem
        return (jnp.where(take, cand, pref), jnp.where(take, rem, rem - c))

    th, rem = lax.fori_loop(0, 32, hib, (jnp.uint32(0), jnp.int32(k)))

    def lob(t, carry):
        plo, rem = carry
        sh = 13 - t
        cand = plo | (jnp.uint32(1) << sh)
        c = jnp.sum(((skey == th) & ((lo >> sh) == (cand >> sh)))
                    .astype(jnp.int32))
        take = c >= rem
        return (jnp.where(take, cand, plo), jnp.where(take, rem, rem - c))

    tl, _ = lax.fori_loop(0, 14, lob, (jnp.uint32(0), rem))
    sel = (skey > th) | ((skey == th) & (lo >= tl))
    self_f = sel.astype(jnp.float32)
    sel_ref[...] = self_f
    gate_ref[...] = jnp.tanh(scr) * self_f


def _tc_call(body, out_shapes, *args):
    return pl.pallas_call(
        body,
        out_shape=out_shapes,
    )(*args)


def kernel(x, edge_index, down_w0, down_b0, down_w1, down_b1, down_w2,
           down_b2, down_w3, down_b3, pool_p0, pool_p1, pool_p2,
           up_w0, up_b0, up_w1, up_b1, up_w2, up_b2):
    f32 = jnp.float32
    rows = edge_index[0]
    cols = edge_index[1]
    rows_fl = rows.reshape(NW, EPW)
    cols_fl = cols.reshape(NW, EPW)
    rows_ch = rows.reshape(16, NCT, CH)
    cols_ch = cols.reshape(16, NCT, CH)
    zrow = jnp.zeros((128, HH), f32)

    xp = jnp.pad(x, ((0, NP - N), (0, 0)))
    m0f = jnp.zeros((NP,), f32).at[:N].set(1.0)
    down_w = [down_w0, down_w1, down_w2, down_w3]
    down_b = [down_b0.reshape(1, H), down_b1.reshape(1, H),
              down_b2.reshape(1, H), down_b3.reshape(1, H)]
    up_w = [up_w0, up_w1, up_w2]
    up_b = [up_b0.reshape(1, H), up_b1.reshape(1, H), up_b2.reshape(1, H)]
    pool_p = [pool_p0.reshape(H, 1), pool_p1.reshape(H, 1),
              pool_p2.reshape(H, 1)]

    def deg(mf):
        sparts = _deg_sc(rows_fl, cols_fl, mf)
        return sparts.T

    def row_full(y):
        y0 = lax.slice(y, (0, 0), (NP, HH))
        y1 = lax.slice(y, (0, HH), (NP, H))
        return _row_sc(y0, y1, rows_ch, cols_ch, zrow)

    def row_dyn(y, ce):
        y0 = lax.slice(y, (0, 0), (NP, HH))
        y1 = lax.slice(y, (0, HH), (NP, H))
        return _row_dyn_sc(y0, y1, ce[0], ce[1], ce[2], zrow)

    def select(scr, mf_prev, k):
        sel2, gate2 = _tc_call(
            functools.partial(_select_body, k),
            (jax.ShapeDtypeStruct((NP // 128, 128), f32),
             jax.ShapeDtypeStruct((NP // 128, 128), f32)),
            scr.reshape(NP // 128, 128), mf_prev.reshape(NP // 128, 128))
        return sel2.reshape(NP), gate2.reshape(NP, 1)

    sds = jax.ShapeDtypeStruct
    def compact(mf):
        rows_t = rows.reshape(16, EPT)
        cols_t = cols.reshape(16, EPT)
        cr, cc, cnts = _compact_sc(rows_t, cols_t, mf)
        cr3 = lax.slice(cr, (0, 0), (16, CAPC * CH)).reshape(16, CAPC, CH)
        cc3 = lax.slice(cc, (0, 0), (16, CAPC * CH)).reshape(16, CAPC, CH)
        return (cr3, cc3, cnts)

    masks_f = [m0f]
    masks_c = [m0f.reshape(NP, 1)]
    hs = []
    dinvs = []
    gate = None
    comp = [None]
    h_in = xp
    scr = None
    n_act = N
    for i in range(DEPTH + 1):
        sparts = deg(masks_f[i])
        if i == 0:
            y, dinv = _tc_call(_prep_first_body,
                               (sds((NP, H), f32), sds((NP, 1), f32)),
                               h_in, down_w[i], sparts, masks_c[i])
        else:
            y, dinv = _tc_call(_prep_gated_body,
                               (sds((NP, H), f32), sds((NP, 1), f32)),
                               h_in, gate, down_w[i], sparts, masks_c[i])
        agg = row_full(y) if i == 0 else row_dyn(y, comp[i])
        dinvs.append(dinv)
        if i == DEPTH:
            agg3, y3 = agg, y
            break
        h, scr = _tc_call(_combine_score_body,
                          (sds((NP, H), f32), sds((NP, 1), f32)),
                          agg, y, dinv, down_b[i], pool_p[i])
        hs.append(h)
        k = (n_act + 1) // 2
        mf, gate = select(scr, masks_f[i], k)
        masks_f.append(mf)
        masks_c.append(mf.reshape(NP, 1))
        n_act = k
        h_in = h
        comp.append(compact(mf))

    agg_p, y_p = agg3, y3
    for i in range(DEPTH):
        j = DEPTH - 1 - i
        b_p = down_b[DEPTH] if i == 0 else up_b[i - 1]
        dinv_p = dinvs[j + 1]
        y_n = _up_fuse_call(agg_p, y_p, dinv_p, b_p, hs[j], masks_c[j + 1],
                            up_w[i], dinvs[j])
        agg_p = row_full(y_n) if j == 0 else row_dyn(y_n, comp[j])
        y_p = y_n
    out = _tc_call(functools.partial(_combine_body, False),
                   sds((NP, H), f32), agg_p, y_p, dinvs[0], up_b[DEPTH - 1])
    return out[:N]
